# Initial kernel scaffold; baseline (speedup 1.0000x reference)
import jax
import jax.numpy as jnp
from jax.experimental import pallas as pl


def kernel(var_feats, con_feats, edge_index, edge_val, params):
    raise NotImplementedError("write your pallas kernel here")


# speedup vs baseline: 1.4192x; 1.4192x over previous
import functools

import jax
import jax.numpy as jnp
import numpy as np
from jax import lax
from jax.experimental import pallas as pl
from jax.experimental.pallas import tpu as pltpu
from jax.experimental.pallas import tpu_sc as plsc

H = 64
N_VAR = 50000
N_CON = 25000
E = 800000
N_PROBES = 16
N_HEADS = 4
CLIP = 5.0
STATIC_VAR_IDX = np.array([0, 1, 2, 3, 4, 5, 6, 19, 20])
DYNAMIC_VAR_IDX = np.array([7, 8, 9, 10, 11, 12, 13, 14, 15, 16, 17, 18])

CH = 128
N_TILES = 16
E_PAD = 819200
EP_TILE = E_PAD // N_TILES
NCH = EP_TILE // CH
VAR_PAD = 50048
CON_PAD = 25088
E_CNT_W = E_PAD // 32
NCH_CNT = E_CNT_W // CH


def _mp_kernel_make(n_src, n_dst_pad):
    rpt = n_dst_pad // N_TILES
    mesh = plsc.VectorSubcoreMesh(core_axis_name="c", subcore_axis_name="s")

    @functools.partial(
        pl.kernel, mesh=mesh,
        compiler_params=pltpu.CompilerParams(use_tc_tiling_on_sc=False),
        out_type=jax.ShapeDtypeStruct((2, n_dst_pad, 32), jnp.float32),
        scratch_types=[
            pltpu.VMEM((1, CH), jnp.int32),
            pltpu.VMEM((1, CH), jnp.int32),
            pltpu.VMEM((1, CH), jnp.float32),
            pltpu.VMEM((CH, 32), jnp.float32),
            pltpu.VMEM((64,), jnp.float32),
            pltpu.VMEM_SHARED((n_dst_pad, 32), jnp.float32),
            pltpu.SemaphoreType.DMA,
        ],
    )
    def k(x_hbm, sidx_hbm, didx_hbm, ev_hbm, gp_hbm, zer_hbm, out_hbm,
          sidx_v, didx_v, ev_v, rows_v, gp_v, acc, sem):
        c = lax.axis_index("c")
        s = lax.axis_index("s")
        pltpu.sync_copy(zer_hbm, acc.at[pl.ds(s * rpt, rpt)])
        plsc.subcore_barrier()
        pltpu.sync_copy(gp_hbm.at[c], gp_v)
        wn0 = gp_v[pl.ds(0, 16)]
        wn1 = gp_v[pl.ds(16, 16)]
        bn0 = gp_v[pl.ds(32, 16)]
        bn1 = gp_v[pl.ds(48, 16)]
        coff = c * n_src
        ebase = s * EP_TILE

        def body(j, carry):
            base = ebase + j * CH
            pltpu.sync_copy(sidx_hbm.at[pl.ds(base, CH)], sidx_v.at[0])
            pltpu.sync_copy(didx_hbm.at[pl.ds(base, CH)], didx_v.at[0])
            pltpu.sync_copy(ev_hbm.at[pl.ds(base, CH)], ev_v.at[0])
            for t in range(CH // 16):
                sl = pl.ds(t * 16, 16)
                sidx_v[0, sl] = sidx_v[0, sl] + coff
            pltpu.async_copy(x_hbm.at[sidx_v.at[0]], rows_v, sem).wait()
            for g in range(CH // 16):
                ev16 = ev_v[0, pl.ds(g * 16, 16)]
                for l in range(16):
                    e = g * 16 + l
                    evs = ev16[l]
                    g0 = 1.0 / (1.0 + jnp.exp(evs * wn0 + bn0))
                    g1 = 1.0 / (1.0 + jnp.exp(evs * wn1 + bn1))
                    rows_v[e, pl.ds(0, 16)] = rows_v[e, pl.ds(0, 16)] * g0
                    rows_v[e, pl.ds(16, 16)] = rows_v[e, pl.ds(16, 16)] * g1
            pltpu.sync_copy(rows_v, acc.at[didx_v.at[0]], add=True)
            return carry

        lax.fori_loop(0, NCH, body, 0)
        plsc.subcore_barrier()
        pltpu.sync_copy(acc.at[pl.ds(s * rpt, rpt)],
                        out_hbm.at[c, pl.ds(s * rpt, rpt)])

    return k


def _cnt_kernel_make(n_dst_pad):
    rpt = n_dst_pad // N_TILES
    mesh = plsc.VectorSubcoreMesh(core_axis_name="c", subcore_axis_name="s")

    @functools.partial(
        pl.kernel, mesh=mesh,
        compiler_params=pltpu.CompilerParams(use_tc_tiling_on_sc=False),
        out_type=jax.ShapeDtypeStruct((2, n_dst_pad, 8), jnp.float32),
        scratch_types=[
            pltpu.VMEM((1, CH), jnp.int32),
            pltpu.VMEM((CH, 8), jnp.float32),
            pltpu.VMEM_SHARED((n_dst_pad, 8), jnp.float32),
        ],
    )
    def k(didx_hbm, ones_hbm, zer_hbm, out_hbm, didx_v, ones_v, acc):
        c = lax.axis_index("c")
        s = lax.axis_index("s")
        pltpu.sync_copy(zer_hbm, acc.at[pl.ds(s * rpt, rpt)])
        plsc.subcore_barrier()
        pltpu.sync_copy(ones_hbm, ones_v)
        w = c * N_TILES + s
        ebase = w * E_CNT_W

        def body(j, carry):
            base = ebase + j * CH
            pltpu.sync_copy(didx_hbm.at[pl.ds(base, CH)], didx_v.at[0])
            pltpu.sync_copy(ones_v, acc.at[didx_v.at[0]], add=True)
            return carry

        lax.fori_loop(0, NCH_CNT, body, 0)
        plsc.subcore_barrier()
        pltpu.sync_copy(acc.at[pl.ds(s * rpt, rpt)],
                        out_hbm.at[c, pl.ds(s * rpt, rpt)])

    return k


def _lin(p, x):
    return x @ p['W'].T + p['b']


def _ln(p, x):
    mu = x.mean(-1, keepdims=True)
    var = x.var(-1, keepdims=True)
    return (x - mu) / jnp.sqrt(var + 1e-05) * p['g'] + p['b']


def _emb(p, x):
    x = jax.nn.relu(_ln(p['n1'], _lin(p['l1'], x)))
    return jax.nn.relu(_ln(p['n2'], _lin(p['l2'], x)))


def _mha(p, x):
    m, d = x.shape
    dh = d // N_HEADS
    qkv = x @ p['in_w'].T + p['in_b']
    q, k, v = jnp.split(qkv, 3, axis=-1)
    q = q.reshape(m, N_HEADS, dh).transpose(1, 0, 2)
    k = k.reshape(m, N_HEADS, dh).transpose(1, 0, 2)
    v = v.reshape(m, N_HEADS, dh).transpose(1, 0, 2)
    a = jax.nn.softmax(q @ k.transpose(0, 2, 1) / np.sqrt(dh), axis=-1)
    o = (a @ v).transpose(1, 0, 2).reshape(m, d)
    return _lin(p['out'], o)


def _block(p, x):
    h = x + _mha(p, _ln(p['sa_norm'], x))
    f = _ln(p['ff_norm'], h)
    return h + _lin(p['ff2'], jax.nn.gelu(_lin(p['ff1'], f), approximate=False))


def _mma(p, sta, dyn):
    d = sta.shape[1]
    K = _lin(p['W_k'], sta)
    Vd = _lin(p['W_v_dyn'], dyn)
    Vs = _lin(p['W_v_sta'], sta)
    S = p['Q_macro'] @ K.T / np.sqrt(d)
    Wm = jax.nn.softmax(S, axis=0)
    Wn = Wm / jnp.clip(Wm.sum(1, keepdims=True), 1e-08, None)
    Hd = _block(p['blk_dyn'], Wn @ Vd)
    Hs = _block(p['blk_sta'], Wn @ Vs)
    fb_d = Wm.T @ _lin(p['W_out_dyn'], Hd)
    fb_s = Wm.T @ _lin(p['W_out_sta'], Hs)
    return fb_d, fb_s


def _norm_var(x):
    out = x.at[:, 19].set(jnp.log1p(jnp.abs(x[:, 19])) * jnp.sign(x[:, 19]))
    cols = jnp.array([0, 7, 8, 9, 12, 14, 19, 20])
    v = out[:, cols]
    vn = jnp.clip((v - v.mean(0)) / (jnp.std(v, axis=0, ddof=1) + 1e-06), -CLIP, CLIP)
    return out.at[:, cols].set(vn)


def _norm_con(x):
    out = x.at[:, 5].set(jnp.log1p(jnp.abs(x[:, 5])) * jnp.sign(x[:, 5]))
    cols = jnp.array([0, 1, 3, 4, 5])
    v = out[:, cols]
    vn = jnp.clip((v - v.mean(0)) / (jnp.std(v, axis=0, ddof=1) + 1e-06), -CLIP, CLIP)
    return out.at[:, cols].set(vn)


def _norm_edge(e):
    return jnp.clip((e - e.mean()) / (jnp.std(e, ddof=1) + 1e-06), -CLIP, CLIP)


def _head_body(fused_ref, w1_ref, b1_ref, w2_ref, b2_ref, out_ref):
    h1 = jax.nn.relu(jnp.dot(fused_ref[...], w1_ref[...],
                             preferred_element_type=jnp.float32) + b1_ref[...])
    out_ref[...] = jnp.dot(h1, w2_ref[...],
                           preferred_element_type=jnp.float32) + b2_ref[...]


def _head_pallas(fused, p1, p2):
    n = fused.shape[0]
    blk = 2000
    out = pl.pallas_call(
        _head_body,
        grid=(n // blk,),
        in_specs=[
            pl.BlockSpec((blk, H), lambda i: (i, 0)),
            pl.BlockSpec((H, H), lambda i: (0, 0)),
            pl.BlockSpec((H,), lambda i: (0,)),
            pl.BlockSpec((H, 1), lambda i: (0, 0)),
            pl.BlockSpec((1,), lambda i: (0,)),
        ],
        out_specs=pl.BlockSpec((blk, 1), lambda i: (i, 0)),
        out_shape=jax.ShapeDtypeStruct((n, 1), jnp.float32),
    )(fused, p1['W'].T, p1['b'], p2['W'].T, p2['b'])
    return out[:, 0]


def _gate_params(p):
    w = p['W'][:, 0]
    b = p['b']
    return jnp.stack([
        jnp.concatenate([-w[0:32], -b[0:32]]),
        jnp.concatenate([-w[32:64], -b[32:64]]),
    ])


class _MP:
    def __init__(self, ci, vi, ev):
        pad = E_PAD - E
        ar = jnp.arange(pad, dtype=jnp.int32)
        self.vi_g = jnp.concatenate([vi, ar % N_VAR])
        self.ci_g = jnp.concatenate([ci, ar % N_CON])
        self.vi_s = jnp.concatenate([vi, N_VAR + (ar % 16)])
        self.ci_s = jnp.concatenate([ci, N_CON + (ar % 16)])
        self.ev = jnp.concatenate([ev, jnp.zeros((pad,), jnp.float32)])
        self.zer_v = jnp.zeros((VAR_PAD // N_TILES, 32), jnp.float32)
        self.zer_c = jnp.zeros((CON_PAD // N_TILES, 32), jnp.float32)
        self.to_con = _mp_kernel_make(N_VAR, CON_PAD)
        self.to_var = _mp_kernel_make(N_CON, VAR_PAD)
        ones = jnp.ones((CH, 8), jnp.float32)
        cnt_c = _cnt_kernel_make(CON_PAD)(
            self.ci_s, ones, jnp.zeros((CON_PAD // N_TILES, 8), jnp.float32))
        cnt_v = _cnt_kernel_make(VAR_PAD)(
            self.vi_s, ones, jnp.zeros((VAR_PAD // N_TILES, 8), jnp.float32))
        self.inv_c = 1.0 / jnp.clip(cnt_c.sum(0)[:N_CON, 0], 1.0, None)
        self.inv_v = 1.0 / jnp.clip(cnt_v.sum(0)[:N_VAR, 0], 1.0, None)

    def v2c(self, x, gate_p):
        x2 = jnp.concatenate([x[:, :32], x[:, 32:]], axis=0)
        out = self.to_con(x2, self.vi_g, self.ci_s, self.ev,
                          _gate_params(gate_p), self.zer_c)
        agg = jnp.concatenate([out[0, :N_CON], out[1, :N_CON]], axis=-1)
        return agg * self.inv_c[:, None]

    def c2v(self, x, gate_p):
        x2 = jnp.concatenate([x[:, :32], x[:, 32:]], axis=0)
        out = self.to_var(x2, self.ci_g, self.vi_s, self.ev,
                          _gate_params(gate_p), self.zer_v)
        agg = jnp.concatenate([out[0, :N_VAR], out[1, :N_VAR]], axis=-1)
        return agg * self.inv_v[:, None]


def _gcn_layer(p, vh, ch, mp):
    agg = mp.v2c(_lin(p['v2c_lin'], vh), p['v2c_gate'])
    ch_new = jax.nn.relu(_ln(p['v2c_ln'], _lin(p['v2c_upd'], jnp.concatenate([agg, ch], -1))))
    agg = mp.c2v(_lin(p['c2v_lin'], ch_new), p['c2v_gate'])
    vh_new = jax.nn.relu(_ln(p['c2v_ln'], _lin(p['c2v_upd'], jnp.concatenate([agg, vh], -1))))
    return vh_new, ch_new


def _gcn(p, vf, cf, mp):
    vh = _emb(p['var_emb'], vf)
    ch = _emb(p['con_emb'], cf)
    for lp in p['layers']:
        dv, dc = _gcn_layer(lp, vh, ch, mp)
        vh = vh + dv
        ch = ch + dc
    return vh, ch


def kernel(var_feats, con_feats, edge_index, edge_val, params):
    vf = _norm_var(var_feats)
    cf = _norm_con(con_feats)
    ev = _norm_edge(edge_val)
    ci, vi = edge_index[0], edge_index[1]
    mp = _MP(ci, vi, ev)
    vh_s, ch_s = _gcn(params['gcn_sta'], vf[:, STATIC_VAR_IDX], cf, mp)
    vh_d, ch_d = _gcn(params['gcn_dyn'], vf[:, DYNAMIC_VAR_IDX], cf, mp)
    fb_d, fb_s = _mma(params['mma'], vh_s, vh_d)
    fused = jax.nn.relu(_ln(params['fuse_ln'], _lin(params['fuse'], jnp.concatenate([vh_s + fb_s, vh_d + fb_d], -1))))
    return _head_pallas(fused, params['head1'], params['head2'])


# speedup vs baseline: 1.6837x; 1.1863x over previous
import functools

import jax
import jax.numpy as jnp
import numpy as np
from jax import lax
from jax.experimental import pallas as pl
from jax.experimental.pallas import tpu as pltpu
from jax.experimental.pallas import tpu_sc as plsc

H = 64
N_VAR = 50000
N_CON = 25000
E = 800000
N_PROBES = 16
N_HEADS = 4
CLIP = 5.0
STATIC_VAR_IDX = np.array([0, 1, 2, 3, 4, 5, 6, 19, 20])
DYNAMIC_VAR_IDX = np.array([7, 8, 9, 10, 11, 12, 13, 14, 15, 16, 17, 18])

CH = 128
GP_C = 8
GP_V = 2
N_TILES = 16
E_PAD = 819200
EP_TILE = E_PAD // N_TILES
VAR_PAD = 50048
CON_PAD = 25088
E_CNT_W = E_PAD // 32
NCH_CNT = E_CNT_W // CH


def _mp_kernel_make(n_src, n_dst_pad, gp):
    rpt = n_dst_pad // N_TILES
    nblk = EP_TILE // (CH * gp)
    mesh = plsc.VectorSubcoreMesh(core_axis_name="c", subcore_axis_name="s")

    @functools.partial(
        pl.kernel, mesh=mesh,
        compiler_params=pltpu.CompilerParams(use_tc_tiling_on_sc=False),
        out_type=jax.ShapeDtypeStruct((2, n_dst_pad, 32), jnp.float32),
        scratch_types=[
            pltpu.VMEM((2, gp, CH), jnp.int32),
            pltpu.VMEM((2, gp, CH), jnp.int32),
            pltpu.VMEM((2, gp, CH), jnp.float32),
            pltpu.VMEM((2, gp, CH, 32), jnp.float32),
            pltpu.VMEM((64,), jnp.float32),
            pltpu.VMEM_SHARED((n_dst_pad, 32), jnp.float32),
            pltpu.SemaphoreType.DMA,
            pltpu.SemaphoreType.DMA,
            pltpu.SemaphoreType.DMA,
            pltpu.SemaphoreType.DMA,
            pltpu.SemaphoreType.DMA,
            pltpu.SemaphoreType.DMA,
        ],
    )
    def k(x_hbm, sidx_hbm, didx_hbm, ev_hbm, gp_hbm, zer_hbm, zidx_hbm,
          out_hbm, sidx_v, didx_v, ev_v, rows_v, gp_v, acc,
          lsem0, lsem1, gsem0, gsem1, ssem0, ssem1):
        lsem = (lsem0, lsem1)
        gsem = (gsem0, gsem1)
        ssem = (ssem0, ssem1)
        c = lax.axis_index("c")
        s = lax.axis_index("s")
        pltpu.sync_copy(zer_hbm, acc.at[pl.ds(s * rpt, rpt)])
        plsc.subcore_barrier()
        pltpu.sync_copy(gp_hbm.at[c], gp_v)
        wn0 = gp_v[pl.ds(0, 16)]
        wn1 = gp_v[pl.ds(16, 16)]
        bn0 = gp_v[pl.ds(32, 16)]
        bn1 = gp_v[pl.ds(48, 16)]
        coff = c * n_src
        bbase = s * nblk

        def fire_loads(b, blk):
            pltpu.async_copy(sidx_hbm.at[blk], sidx_v.at[b], lsem[b])
            pltpu.async_copy(didx_hbm.at[blk], didx_v.at[b], lsem[b])
            pltpu.async_copy(ev_hbm.at[blk], ev_v.at[b], lsem[b])

        def drain_loads(b, blk):
            pltpu.make_async_copy(sidx_hbm.at[blk], sidx_v.at[b], lsem[b]).wait()
            pltpu.make_async_copy(didx_hbm.at[blk], didx_v.at[b], lsem[b]).wait()
            pltpu.make_async_copy(ev_hbm.at[blk], ev_v.at[b], lsem[b]).wait()

        def offset_sidx(b):
            for kk in range(gp):
                for t in range(CH // 16):
                    sl = pl.ds(t * 16, 16)
                    sidx_v[b, kk, sl] = sidx_v[b, kk, sl] + coff

        def fire_gathers(b):
            for kk in range(gp):
                pltpu.async_copy(x_hbm.at[sidx_v.at[b, kk]],
                                 rows_v.at[b, kk], gsem[b])

        def drain_gathers(b):
            for kk in range(gp):
                pltpu.make_async_copy(x_hbm.at[sidx_v.at[b, kk]],
                                      rows_v.at[b, kk], gsem[b]).wait()

        def drain_scatters(b):
            for kk in range(gp):
                pltpu.make_async_copy(rows_v.at[b, kk],
                                      acc.at[didx_v.at[b, kk]], ssem[b]).wait()

        def compute_and_scatter(b):
            def cbody(kk, carry):
                for g in range(CH // 16):
                    ev16 = ev_v[b, kk, pl.ds(g * 16, 16)]
                    for l in range(16):
                        e = g * 16 + l
                        evs = ev16[l]
                        g0 = 1.0 / (1.0 + jnp.exp(evs * wn0 + bn0))
                        g1 = 1.0 / (1.0 + jnp.exp(evs * wn1 + bn1))
                        rows_v[b, kk, e, pl.ds(0, 16)] = (
                            rows_v[b, kk, e, pl.ds(0, 16)] * g0)
                        rows_v[b, kk, e, pl.ds(16, 16)] = (
                            rows_v[b, kk, e, pl.ds(16, 16)] * g1)
                pltpu.async_copy(rows_v.at[b, kk], acc.at[didx_v.at[b, kk]],
                                 ssem[b], add=True)
                return carry
            lax.fori_loop(0, gp, cbody, 0)

        pltpu.sync_copy(zidx_hbm, didx_v.at[1])
        for kk in range(gp):
            pltpu.sync_copy(zer_hbm.at[pl.ds(0, CH)], rows_v.at[1, kk])
        for kk in range(gp):
            pltpu.async_copy(rows_v.at[1, kk], acc.at[didx_v.at[1, kk]],
                             ssem[1], add=True)
        fire_loads(0, bbase)
        drain_loads(0, bbase)
        offset_sidx(0)
        fire_gathers(0)

        def iteration(j, b):
            bn = 1 - b
            blkn = bbase + jnp.minimum(j + 1, nblk - 1)
            drain_scatters(bn)
            fire_loads(bn, blkn)
            drain_loads(bn, blkn)
            offset_sidx(bn)
            fire_gathers(bn)
            drain_gathers(b)
            compute_and_scatter(b)

        def obody(jj, carry):
            iteration(2 * jj, 0)
            iteration(2 * jj + 1, 1)
            return carry

        lax.fori_loop(0, nblk // 2, obody, 0)
        drain_gathers(0)
        drain_scatters(1)
        plsc.subcore_barrier()
        pltpu.sync_copy(acc.at[pl.ds(s * rpt, rpt)],
                        out_hbm.at[c, pl.ds(s * rpt, rpt)])

    return k


def _cnt_kernel_make(n_dst_pad, gp):
    rpt = n_dst_pad // N_TILES
    mesh = plsc.VectorSubcoreMesh(core_axis_name="c", subcore_axis_name="s")

    @functools.partial(
        pl.kernel, mesh=mesh,
        compiler_params=pltpu.CompilerParams(use_tc_tiling_on_sc=False),
        out_type=jax.ShapeDtypeStruct((2, n_dst_pad, 8), jnp.float32),
        scratch_types=[
            pltpu.VMEM((gp, CH), jnp.int32),
            pltpu.VMEM((CH, 8), jnp.float32),
            pltpu.VMEM_SHARED((n_dst_pad, 8), jnp.float32),
        ],
    )
    def k(didx_hbm, ones_hbm, zer_hbm, out_hbm, didx_v, ones_v, acc):
        c = lax.axis_index("c")
        s = lax.axis_index("s")
        pltpu.sync_copy(zer_hbm, acc.at[pl.ds(s * rpt, rpt)])
        plsc.subcore_barrier()
        pltpu.sync_copy(ones_hbm, ones_v)
        w = c * N_TILES + s
        nblk_w = E_PAD // (CH * gp) // 32
        bbase = w * nblk_w

        def body(j, carry):
            pltpu.sync_copy(didx_hbm.at[bbase + j], didx_v)
            for kk in range(gp):
                pltpu.sync_copy(ones_v, acc.at[didx_v.at[kk]], add=True)
            return carry

        lax.fori_loop(0, nblk_w, body, 0)
        plsc.subcore_barrier()
        pltpu.sync_copy(acc.at[pl.ds(s * rpt, rpt)],
                        out_hbm.at[c, pl.ds(s * rpt, rpt)])

    return k


def _lin(p, x):
    return x @ p['W'].T + p['b']


def _ln(p, x):
    mu = x.mean(-1, keepdims=True)
    var = x.var(-1, keepdims=True)
    return (x - mu) / jnp.sqrt(var + 1e-05) * p['g'] + p['b']


def _emb(p, x):
    x = jax.nn.relu(_ln(p['n1'], _lin(p['l1'], x)))
    return jax.nn.relu(_ln(p['n2'], _lin(p['l2'], x)))


def _mha(p, x):
    m, d = x.shape
    dh = d // N_HEADS
    qkv = x @ p['in_w'].T + p['in_b']
    q, k, v = jnp.split(qkv, 3, axis=-1)
    q = q.reshape(m, N_HEADS, dh).transpose(1, 0, 2)
    k = k.reshape(m, N_HEADS, dh).transpose(1, 0, 2)
    v = v.reshape(m, N_HEADS, dh).transpose(1, 0, 2)
    a = jax.nn.softmax(q @ k.transpose(0, 2, 1) / np.sqrt(dh), axis=-1)
    o = (a @ v).transpose(1, 0, 2).reshape(m, d)
    return _lin(p['out'], o)


def _block(p, x):
    h = x + _mha(p, _ln(p['sa_norm'], x))
    f = _ln(p['ff_norm'], h)
    return h + _lin(p['ff2'], jax.nn.gelu(_lin(p['ff1'], f), approximate=False))


def _mma(p, sta, dyn):
    d = sta.shape[1]
    K = _lin(p['W_k'], sta)
    Vd = _lin(p['W_v_dyn'], dyn)
    Vs = _lin(p['W_v_sta'], sta)
    S = p['Q_macro'] @ K.T / np.sqrt(d)
    Wm = jax.nn.softmax(S, axis=0)
    Wn = Wm / jnp.clip(Wm.sum(1, keepdims=True), 1e-08, None)
    Hd = _block(p['blk_dyn'], Wn @ Vd)
    Hs = _block(p['blk_sta'], Wn @ Vs)
    fb_d = Wm.T @ _lin(p['W_out_dyn'], Hd)
    fb_s = Wm.T @ _lin(p['W_out_sta'], Hs)
    return fb_d, fb_s


def _norm_var(x):
    out = x.at[:, 19].set(jnp.log1p(jnp.abs(x[:, 19])) * jnp.sign(x[:, 19]))
    cols = jnp.array([0, 7, 8, 9, 12, 14, 19, 20])
    v = out[:, cols]
    vn = jnp.clip((v - v.mean(0)) / (jnp.std(v, axis=0, ddof=1) + 1e-06), -CLIP, CLIP)
    return out.at[:, cols].set(vn)


def _norm_con(x):
    out = x.at[:, 5].set(jnp.log1p(jnp.abs(x[:, 5])) * jnp.sign(x[:, 5]))
    cols = jnp.array([0, 1, 3, 4, 5])
    v = out[:, cols]
    vn = jnp.clip((v - v.mean(0)) / (jnp.std(v, axis=0, ddof=1) + 1e-06), -CLIP, CLIP)
    return out.at[:, cols].set(vn)


def _norm_edge(e):
    return jnp.clip((e - e.mean()) / (jnp.std(e, ddof=1) + 1e-06), -CLIP, CLIP)


def _head_body(fused_ref, w1_ref, b1_ref, w2_ref, b2_ref, out_ref):
    h1 = jax.nn.relu(jnp.dot(fused_ref[...], w1_ref[...],
                             preferred_element_type=jnp.float32) + b1_ref[...])
    out_ref[...] = jnp.dot(h1, w2_ref[...],
                           preferred_element_type=jnp.float32) + b2_ref[...]


def _head_pallas(fused, p1, p2):
    n = fused.shape[0]
    blk = 2000
    out = pl.pallas_call(
        _head_body,
        grid=(n // blk,),
        in_specs=[
            pl.BlockSpec((blk, H), lambda i: (i, 0)),
            pl.BlockSpec((H, H), lambda i: (0, 0)),
            pl.BlockSpec((H,), lambda i: (0,)),
            pl.BlockSpec((H, 1), lambda i: (0, 0)),
            pl.BlockSpec((1,), lambda i: (0,)),
        ],
        out_specs=pl.BlockSpec((blk, 1), lambda i: (i, 0)),
        out_shape=jax.ShapeDtypeStruct((n, 1), jnp.float32),
    )(fused, p1['W'].T, p1['b'], p2['W'].T, p2['b'])
    return out[:, 0]


def _gate_params(p):
    w = p['W'][:, 0]
    b = p['b']
    return jnp.stack([
        jnp.concatenate([-w[0:32], -b[0:32]]),
        jnp.concatenate([-w[32:64], -b[32:64]]),
    ])


class _MP:
    def __init__(self, ci, vi, ev):
        pad = E_PAD - E
        ar = jnp.arange(pad, dtype=jnp.int32)
        rs = lambda a, gp: a.reshape(E_PAD // (CH * gp), gp, CH)
        evp = jnp.concatenate([ev, jnp.zeros((pad,), jnp.float32)])
        self.vi_g = rs(jnp.concatenate([vi, ar % N_VAR]), GP_C)
        self.ci_g = rs(jnp.concatenate([ci, ar % N_CON]), GP_V)
        self.vi_s = rs(jnp.concatenate([vi, N_VAR + (ar % 16)]), GP_V)
        self.ci_s = rs(jnp.concatenate([ci, N_CON + (ar % 16)]), GP_C)
        self.ev_c = rs(evp, GP_C)
        self.ev_v = rs(evp, GP_V)
        self.zidx_c = jnp.zeros((GP_C, CH), jnp.int32)
        self.zidx_v = jnp.zeros((GP_V, CH), jnp.int32)
        self.zer_v = jnp.zeros((VAR_PAD // N_TILES, 32), jnp.float32)
        self.zer_c = jnp.zeros((CON_PAD // N_TILES, 32), jnp.float32)
        self.to_con = _mp_kernel_make(N_VAR, CON_PAD, GP_C)
        self.to_var = _mp_kernel_make(N_CON, VAR_PAD, GP_V)
        ones = jnp.ones((CH, 8), jnp.float32)
        cnt_c = _cnt_kernel_make(CON_PAD, GP_C)(
            self.ci_s, ones, jnp.zeros((CON_PAD // N_TILES, 8), jnp.float32))
        cnt_v = _cnt_kernel_make(VAR_PAD, GP_V)(
            self.vi_s, ones, jnp.zeros((VAR_PAD // N_TILES, 8), jnp.float32))
        self.inv_c = 1.0 / jnp.clip(cnt_c.sum(0)[:N_CON, 0], 1.0, None)
        self.inv_v = 1.0 / jnp.clip(cnt_v.sum(0)[:N_VAR, 0], 1.0, None)

    def v2c(self, x, gate_p):
        x2 = jnp.concatenate([x[:, :32], x[:, 32:]], axis=0)
        out = self.to_con(x2, self.vi_g, self.ci_s, self.ev_c,
                          _gate_params(gate_p), self.zer_c, self.zidx_c)
        agg = jnp.concatenate([out[0, :N_CON], out[1, :N_CON]], axis=-1)
        return agg * self.inv_c[:, None]

    def c2v(self, x, gate_p):
        x2 = jnp.concatenate([x[:, :32], x[:, 32:]], axis=0)
        out = self.to_var(x2, self.ci_g, self.vi_s, self.ev_v,
                          _gate_params(gate_p), self.zer_v, self.zidx_v)
        agg = jnp.concatenate([out[0, :N_VAR], out[1, :N_VAR]], axis=-1)
        return agg * self.inv_v[:, None]


def _gcn_layer(p, vh, ch, mp):
    agg = mp.v2c(_lin(p['v2c_lin'], vh), p['v2c_gate'])
    ch_new = jax.nn.relu(_ln(p['v2c_ln'], _lin(p['v2c_upd'], jnp.concatenate([agg, ch], -1))))
    agg = mp.c2v(_lin(p['c2v_lin'], ch_new), p['c2v_gate'])
    vh_new = jax.nn.relu(_ln(p['c2v_ln'], _lin(p['c2v_upd'], jnp.concatenate([agg, vh], -1))))
    return vh_new, ch_new


def _gcn(p, vf, cf, mp):
    vh = _emb(p['var_emb'], vf)
    ch = _emb(p['con_emb'], cf)
    for lp in p['layers']:
        dv, dc = _gcn_layer(lp, vh, ch, mp)
        vh = vh + dv
        ch = ch + dc
    return vh, ch


def kernel(var_feats, con_feats, edge_index, edge_val, params):
    vf = _norm_var(var_feats)
    cf = _norm_con(con_feats)
    ev = _norm_edge(edge_val)
    ci, vi = edge_index[0], edge_index[1]
    mp = _MP(ci, vi, ev)
    vh_s, ch_s = _gcn(params['gcn_sta'], vf[:, STATIC_VAR_IDX], cf, mp)
    vh_d, ch_d = _gcn(params['gcn_dyn'], vf[:, DYNAMIC_VAR_IDX], cf, mp)
    fb_d, fb_s = _mma(params['mma'], vh_s, vh_d)
    fused = jax.nn.relu(_ln(params['fuse_ln'], _lin(params['fuse'], jnp.concatenate([vh_s + fb_s, vh_d + fb_d], -1))))
    return _head_pallas(fused, params['head1'], params['head2'])


# speedup vs baseline: 2.0708x; 1.2299x over previous
import functools

import jax
import jax.numpy as jnp
import numpy as np
from jax import lax
from jax.experimental import pallas as pl
from jax.experimental.pallas import tpu as pltpu
from jax.experimental.pallas import tpu_sc as plsc

H = 64
N_VAR = 50000
N_CON = 25000
E = 800000
N_PROBES = 16
N_HEADS = 4
CLIP = 5.0
STATIC_VAR_IDX = np.array([0, 1, 2, 3, 4, 5, 6, 19, 20])
DYNAMIC_VAR_IDX = np.array([7, 8, 9, 10, 11, 12, 13, 14, 15, 16, 17, 18])

CH = 128
GP_C = 8
GP_V = 2
N_TILES = 16
E_PAD = 819200
EP_TILE = E_PAD // N_TILES
VAR_PAD = 50048
CON_PAD = 25088
E_CNT_W = E_PAD // 32
NCH_CNT = E_CNT_W // CH


def _mp_kernel_make(n_src, n_dst_pad, gp):
    rpt = n_dst_pad // N_TILES
    nblk = EP_TILE // (CH * gp)
    mesh = plsc.VectorSubcoreMesh(core_axis_name="c", subcore_axis_name="s")

    @functools.partial(
        pl.kernel, mesh=mesh,
        compiler_params=pltpu.CompilerParams(use_tc_tiling_on_sc=False),
        out_type=jax.ShapeDtypeStruct((2, n_dst_pad, 32), jnp.float32),
        scratch_types=[
            pltpu.VMEM((2, gp, CH), jnp.int32),
            pltpu.VMEM((2, gp, CH), jnp.int32),
            pltpu.VMEM((2, gp, CH), jnp.float32),
            pltpu.VMEM((2, gp, CH, 32), jnp.float32),
            pltpu.VMEM((128, 64), jnp.float32),
            pltpu.VMEM_SHARED((n_dst_pad, 32), jnp.float32),
            pltpu.SemaphoreType.DMA,
            pltpu.SemaphoreType.DMA,
            pltpu.SemaphoreType.DMA,
            pltpu.SemaphoreType.DMA,
            pltpu.SemaphoreType.DMA,
            pltpu.SemaphoreType.DMA,
        ],
    )
    def k(x_hbm, sidx_hbm, didx_hbm, ev_hbm, tab_hbm, zer_hbm, zidx_hbm,
          out_hbm, sidx_v, didx_v, ev_v, rows_v, tab_v, acc,
          lsem0, lsem1, gsem0, gsem1, ssem0, ssem1):
        lsem = (lsem0, lsem1)
        gsem = (gsem0, gsem1)
        ssem = (ssem0, ssem1)
        c = lax.axis_index("c")
        s = lax.axis_index("s")
        pltpu.sync_copy(zer_hbm, acc.at[pl.ds(s * rpt, rpt)])
        plsc.subcore_barrier()
        pltpu.sync_copy(tab_hbm.at[c], tab_v)
        coff = c * n_src
        bbase = s * nblk

        def fire_loads(b, blk):
            pltpu.async_copy(sidx_hbm.at[blk], sidx_v.at[b], lsem[b])
            pltpu.async_copy(didx_hbm.at[blk], didx_v.at[b], lsem[b])
            pltpu.async_copy(ev_hbm.at[blk], ev_v.at[b], lsem[b])

        def drain_loads(b, blk):
            pltpu.make_async_copy(sidx_hbm.at[blk], sidx_v.at[b], lsem[b]).wait()
            pltpu.make_async_copy(didx_hbm.at[blk], didx_v.at[b], lsem[b]).wait()
            pltpu.make_async_copy(ev_hbm.at[blk], ev_v.at[b], lsem[b]).wait()

        def offset_sidx(b):
            for kk in range(gp):
                for t in range(CH // 16):
                    sl = pl.ds(t * 16, 16)
                    sidx_v[b, kk, sl] = sidx_v[b, kk, sl] + coff

        def fire_gathers(b):
            for kk in range(gp):
                pltpu.async_copy(x_hbm.at[sidx_v.at[b, kk]],
                                 rows_v.at[b, kk], gsem[b])

        def drain_gathers(b):
            for kk in range(gp):
                pltpu.make_async_copy(x_hbm.at[sidx_v.at[b, kk]],
                                      rows_v.at[b, kk], gsem[b]).wait()

        def drain_scatters(b):
            for kk in range(gp):
                pltpu.make_async_copy(rows_v.at[b, kk],
                                      acc.at[didx_v.at[b, kk]], ssem[b]).wait()

        def compute_and_scatter(b):
            def cbody(kk, carry):
                for g in range(CH // 16):
                    qf16 = ev_v[b, kk, pl.ds(g * 16, 16)]
                    qi16 = qf16.astype(jnp.int32)
                    fr16 = qf16 - qi16.astype(jnp.float32)
                    for l in range(16):
                        e = g * 16 + l
                        q = qi16[l]
                        f = fr16[l]
                        g0 = (tab_v[q, pl.ds(0, 16)]
                              + f * tab_v[q, pl.ds(32, 16)])
                        g1 = (tab_v[q, pl.ds(16, 16)]
                              + f * tab_v[q, pl.ds(48, 16)])
                        rows_v[b, kk, e, pl.ds(0, 16)] = (
                            rows_v[b, kk, e, pl.ds(0, 16)] * g0)
                        rows_v[b, kk, e, pl.ds(16, 16)] = (
                            rows_v[b, kk, e, pl.ds(16, 16)] * g1)
                pltpu.async_copy(rows_v.at[b, kk], acc.at[didx_v.at[b, kk]],
                                 ssem[b], add=True)
                return carry
            lax.fori_loop(0, gp, cbody, 0)

        pltpu.sync_copy(zidx_hbm, didx_v.at[1])
        for kk in range(gp):
            pltpu.sync_copy(zer_hbm.at[pl.ds(0, CH)], rows_v.at[1, kk])
        for kk in range(gp):
            pltpu.async_copy(rows_v.at[1, kk], acc.at[didx_v.at[1, kk]],
                             ssem[1], add=True)
        fire_loads(0, bbase)
        drain_loads(0, bbase)
        offset_sidx(0)
        fire_gathers(0)

        def iteration(j, b):
            bn = 1 - b
            blkn = bbase + jnp.minimum(j + 1, nblk - 1)
            drain_scatters(bn)
            fire_loads(bn, blkn)
            drain_loads(bn, blkn)
            offset_sidx(bn)
            fire_gathers(bn)
            drain_gathers(b)
            compute_and_scatter(b)

        def obody(jj, carry):
            iteration(2 * jj, 0)
            iteration(2 * jj + 1, 1)
            return carry

        lax.fori_loop(0, nblk // 2, obody, 0)
        drain_gathers(0)
        drain_scatters(1)
        plsc.subcore_barrier()
        pltpu.sync_copy(acc.at[pl.ds(s * rpt, rpt)],
                        out_hbm.at[c, pl.ds(s * rpt, rpt)])

    return k


def _cnt_kernel_make(n_dst_pad, gp):
    rpt = n_dst_pad // N_TILES
    mesh = plsc.VectorSubcoreMesh(core_axis_name="c", subcore_axis_name="s")

    @functools.partial(
        pl.kernel, mesh=mesh,
        compiler_params=pltpu.CompilerParams(use_tc_tiling_on_sc=False),
        out_type=jax.ShapeDtypeStruct((2, n_dst_pad, 8), jnp.float32),
        scratch_types=[
            pltpu.VMEM((gp, CH), jnp.int32),
            pltpu.VMEM((CH, 8), jnp.float32),
            pltpu.VMEM_SHARED((n_dst_pad, 8), jnp.float32),
        ],
    )
    def k(didx_hbm, ones_hbm, zer_hbm, out_hbm, didx_v, ones_v, acc):
        c = lax.axis_index("c")
        s = lax.axis_index("s")
        pltpu.sync_copy(zer_hbm, acc.at[pl.ds(s * rpt, rpt)])
        plsc.subcore_barrier()
        pltpu.sync_copy(ones_hbm, ones_v)
        w = c * N_TILES + s
        nblk_w = E_PAD // (CH * gp) // 32
        bbase = w * nblk_w

        def body(j, carry):
            pltpu.sync_copy(didx_hbm.at[bbase + j], didx_v)
            for kk in range(gp):
                pltpu.sync_copy(ones_v, acc.at[didx_v.at[kk]], add=True)
            return carry

        lax.fori_loop(0, nblk_w, body, 0)
        plsc.subcore_barrier()
        pltpu.sync_copy(acc.at[pl.ds(s * rpt, rpt)],
                        out_hbm.at[c, pl.ds(s * rpt, rpt)])

    return k


def _lin(p, x):
    return x @ p['W'].T + p['b']


def _ln(p, x):
    mu = x.mean(-1, keepdims=True)
    var = x.var(-1, keepdims=True)
    return (x - mu) / jnp.sqrt(var + 1e-05) * p['g'] + p['b']


def _emb(p, x):
    x = jax.nn.relu(_ln(p['n1'], _lin(p['l1'], x)))
    return jax.nn.relu(_ln(p['n2'], _lin(p['l2'], x)))


def _mha(p, x):
    m, d = x.shape
    dh = d // N_HEADS
    qkv = x @ p['in_w'].T + p['in_b']
    q, k, v = jnp.split(qkv, 3, axis=-1)
    q = q.reshape(m, N_HEADS, dh).transpose(1, 0, 2)
    k = k.reshape(m, N_HEADS, dh).transpose(1, 0, 2)
    v = v.reshape(m, N_HEADS, dh).transpose(1, 0, 2)
    a = jax.nn.softmax(q @ k.transpose(0, 2, 1) / np.sqrt(dh), axis=-1)
    o = (a @ v).transpose(1, 0, 2).reshape(m, d)
    return _lin(p['out'], o)


def _block(p, x):
    h = x + _mha(p, _ln(p['sa_norm'], x))
    f = _ln(p['ff_norm'], h)
    return h + _lin(p['ff2'], jax.nn.gelu(_lin(p['ff1'], f), approximate=False))


def _mma(p, sta, dyn):
    d = sta.shape[1]
    K = _lin(p['W_k'], sta)
    Vd = _lin(p['W_v_dyn'], dyn)
    Vs = _lin(p['W_v_sta'], sta)
    S = p['Q_macro'] @ K.T / np.sqrt(d)
    Wm = jax.nn.softmax(S, axis=0)
    Wn = Wm / jnp.clip(Wm.sum(1, keepdims=True), 1e-08, None)
    Hd = _block(p['blk_dyn'], Wn @ Vd)
    Hs = _block(p['blk_sta'], Wn @ Vs)
    fb_d = Wm.T @ _lin(p['W_out_dyn'], Hd)
    fb_s = Wm.T @ _lin(p['W_out_sta'], Hs)
    return fb_d, fb_s


def _norm_var(x):
    out = x.at[:, 19].set(jnp.log1p(jnp.abs(x[:, 19])) * jnp.sign(x[:, 19]))
    cols = jnp.array([0, 7, 8, 9, 12, 14, 19, 20])
    v = out[:, cols]
    vn = jnp.clip((v - v.mean(0)) / (jnp.std(v, axis=0, ddof=1) + 1e-06), -CLIP, CLIP)
    return out.at[:, cols].set(vn)


def _norm_con(x):
    out = x.at[:, 5].set(jnp.log1p(jnp.abs(x[:, 5])) * jnp.sign(x[:, 5]))
    cols = jnp.array([0, 1, 3, 4, 5])
    v = out[:, cols]
    vn = jnp.clip((v - v.mean(0)) / (jnp.std(v, axis=0, ddof=1) + 1e-06), -CLIP, CLIP)
    return out.at[:, cols].set(vn)


def _norm_edge(e):
    return jnp.clip((e - e.mean()) / (jnp.std(e, ddof=1) + 1e-06), -CLIP, CLIP)


def _head_body(fused_ref, w1_ref, b1_ref, w2_ref, b2_ref, out_ref):
    h1 = jax.nn.relu(jnp.dot(fused_ref[...], w1_ref[...],
                             preferred_element_type=jnp.float32) + b1_ref[...])
    out_ref[...] = jnp.dot(h1, w2_ref[...],
                           preferred_element_type=jnp.float32) + b2_ref[...]


def _head_pallas(fused, p1, p2):
    n = fused.shape[0]
    blk = 2000
    out = pl.pallas_call(
        _head_body,
        grid=(n // blk,),
        in_specs=[
            pl.BlockSpec((blk, H), lambda i: (i, 0)),
            pl.BlockSpec((H, H), lambda i: (0, 0)),
            pl.BlockSpec((H,), lambda i: (0,)),
            pl.BlockSpec((H, 1), lambda i: (0, 0)),
            pl.BlockSpec((1,), lambda i: (0,)),
        ],
        out_specs=pl.BlockSpec((blk, 1), lambda i: (i, 0)),
        out_shape=jax.ShapeDtypeStruct((n, 1), jnp.float32),
    )(fused, p1['W'].T, p1['b'], p2['W'].T, p2['b'])
    return out[:, 0]


N_BINS = 128
EV_LO = -5.0
EV_STEP = 10.0 / (N_BINS - 1)


def _gate_table(p):
    w = p['W'][:, 0]
    b = p['b']
    grid = EV_LO + EV_STEP * jnp.arange(N_BINS, dtype=jnp.float32)
    t = jax.nn.sigmoid(grid[:, None] * w[None, :] + b[None, :])
    dt = jnp.concatenate([t[1:] - t[:-1], jnp.zeros((1, H), jnp.float32)], 0)
    return jnp.stack([
        jnp.concatenate([t[:, 0:32], dt[:, 0:32]], axis=1),
        jnp.concatenate([t[:, 32:64], dt[:, 32:64]], axis=1),
    ])


class _MP:
    def __init__(self, ci, vi, ev):
        pad = E_PAD - E
        ar = jnp.arange(pad, dtype=jnp.int32)
        rs = lambda a, gp: a.reshape(E_PAD // (CH * gp), gp, CH)
        qf = jnp.clip((ev - EV_LO) / EV_STEP, 0.0, N_BINS - 1.001)
        evp = jnp.concatenate([qf, jnp.full((pad,), 63.5, jnp.float32)])
        self.vi_g = rs(jnp.concatenate([vi, ar % N_VAR]), GP_C)
        self.ci_g = rs(jnp.concatenate([ci, ar % N_CON]), GP_V)
        self.vi_s = rs(jnp.concatenate([vi, N_VAR + (ar % 16)]), GP_V)
        self.ci_s = rs(jnp.concatenate([ci, N_CON + (ar % 16)]), GP_C)
        self.ev_c = rs(evp, GP_C)
        self.ev_v = rs(evp, GP_V)
        self.zidx_c = jnp.zeros((GP_C, CH), jnp.int32)
        self.zidx_v = jnp.zeros((GP_V, CH), jnp.int32)
        self.zer_v = jnp.zeros((VAR_PAD // N_TILES, 32), jnp.float32)
        self.zer_c = jnp.zeros((CON_PAD // N_TILES, 32), jnp.float32)
        self.to_con = _mp_kernel_make(N_VAR, CON_PAD, GP_C)
        self.to_var = _mp_kernel_make(N_CON, VAR_PAD, GP_V)
        ones = jnp.ones((CH, 8), jnp.float32)
        cnt_c = _cnt_kernel_make(CON_PAD, GP_C)(
            self.ci_s, ones, jnp.zeros((CON_PAD // N_TILES, 8), jnp.float32))
        cnt_v = _cnt_kernel_make(VAR_PAD, GP_V)(
            self.vi_s, ones, jnp.zeros((VAR_PAD // N_TILES, 8), jnp.float32))
        self.inv_c = 1.0 / jnp.clip(cnt_c.sum(0)[:N_CON, 0], 1.0, None)
        self.inv_v = 1.0 / jnp.clip(cnt_v.sum(0)[:N_VAR, 0], 1.0, None)

    def v2c(self, x, gate_p):
        x2 = jnp.concatenate([x[:, :32], x[:, 32:]], axis=0)
        out = self.to_con(x2, self.vi_g, self.ci_s, self.ev_c,
                          _gate_table(gate_p), self.zer_c, self.zidx_c)
        agg = jnp.concatenate([out[0, :N_CON], out[1, :N_CON]], axis=-1)
        return agg * self.inv_c[:, None]

    def c2v(self, x, gate_p):
        x2 = jnp.concatenate([x[:, :32], x[:, 32:]], axis=0)
        out = self.to_var(x2, self.ci_g, self.vi_s, self.ev_v,
                          _gate_table(gate_p), self.zer_v, self.zidx_v)
        agg = jnp.concatenate([out[0, :N_VAR], out[1, :N_VAR]], axis=-1)
        return agg * self.inv_v[:, None]


def _gcn_layer(p, vh, ch, mp):
    agg = mp.v2c(_lin(p['v2c_lin'], vh), p['v2c_gate'])
    ch_new = jax.nn.relu(_ln(p['v2c_ln'], _lin(p['v2c_upd'], jnp.concatenate([agg, ch], -1))))
    agg = mp.c2v(_lin(p['c2v_lin'], ch_new), p['c2v_gate'])
    vh_new = jax.nn.relu(_ln(p['c2v_ln'], _lin(p['c2v_upd'], jnp.concatenate([agg, vh], -1))))
    return vh_new, ch_new


def _gcn(p, vf, cf, mp):
    vh = _emb(p['var_emb'], vf)
    ch = _emb(p['con_emb'], cf)
    for lp in p['layers']:
        dv, dc = _gcn_layer(lp, vh, ch, mp)
        vh = vh + dv
        ch = ch + dc
    return vh, ch


def kernel(var_feats, con_feats, edge_index, edge_val, params):
    vf = _norm_var(var_feats)
    cf = _norm_con(con_feats)
    ev = _norm_edge(edge_val)
    ci, vi = edge_index[0], edge_index[1]
    mp = _MP(ci, vi, ev)
    vh_s, ch_s = _gcn(params['gcn_sta'], vf[:, STATIC_VAR_IDX], cf, mp)
    vh_d, ch_d = _gcn(params['gcn_dyn'], vf[:, DYNAMIC_VAR_IDX], cf, mp)
    fb_d, fb_s = _mma(params['mma'], vh_s, vh_d)
    fused = jax.nn.relu(_ln(params['fuse_ln'], _lin(params['fuse'], jnp.concatenate([vh_s + fb_s, vh_d + fb_d], -1))))
    return _head_pallas(fused, params['head1'], params['head2'])


# speedup vs baseline: 2.0709x; 1.0001x over previous
import functools

import jax
import jax.numpy as jnp
import numpy as np
from jax import lax
from jax.experimental import pallas as pl
from jax.experimental.pallas import tpu as pltpu
from jax.experimental.pallas import tpu_sc as plsc

H = 64
N_VAR = 50000
N_CON = 25000
E = 800000
N_PROBES = 16
N_HEADS = 4
CLIP = 5.0
STATIC_VAR_IDX = np.array([0, 1, 2, 3, 4, 5, 6, 19, 20])
DYNAMIC_VAR_IDX = np.array([7, 8, 9, 10, 11, 12, 13, 14, 15, 16, 17, 18])

CH = 128
GP_C = 8
GP_V = 2
N_TILES = 16
E_PAD = 819200
EP_TILE = E_PAD // N_TILES
VAR_PAD = 50048
CON_PAD = 25088
E_CNT_W = E_PAD // 32
NCH_CNT = E_CNT_W // CH


def _mp_kernel_make(n_src, n_dst_pad, gp):
    rpt = n_dst_pad // N_TILES
    nblk = EP_TILE // (CH * gp)
    mesh = plsc.VectorSubcoreMesh(core_axis_name="c", subcore_axis_name="s")

    @functools.partial(
        pl.kernel, mesh=mesh,
        compiler_params=pltpu.CompilerParams(use_tc_tiling_on_sc=False),
        out_type=jax.ShapeDtypeStruct((2, n_dst_pad, 32), jnp.float32),
        scratch_types=[
            pltpu.VMEM((2, gp, CH), jnp.int32),
            pltpu.VMEM((2, gp, CH), jnp.int32),
            pltpu.VMEM((2, gp, CH), jnp.float32),
            pltpu.VMEM((2, gp, CH, 32), jnp.float32),
            pltpu.VMEM((128, 64), jnp.float32),
            pltpu.VMEM_SHARED((n_dst_pad, 32), jnp.float32),
            pltpu.SemaphoreType.DMA,
            pltpu.SemaphoreType.DMA,
            pltpu.SemaphoreType.DMA,
            pltpu.SemaphoreType.DMA,
            pltpu.SemaphoreType.DMA,
            pltpu.SemaphoreType.DMA,
        ],
    )
    def k(x_hbm, sidx_hbm, didx_hbm, ev_hbm, tab_hbm, zer_hbm, zidx_hbm,
          out_hbm, sidx_v, didx_v, ev_v, rows_v, tab_v, acc,
          lsem0, lsem1, gsem0, gsem1, ssem0, ssem1):
        lsem = (lsem0, lsem1)
        gsem = (gsem0, gsem1)
        ssem = (ssem0, ssem1)
        c = lax.axis_index("c")
        s = lax.axis_index("s")
        pltpu.sync_copy(zer_hbm, acc.at[pl.ds(s * rpt, rpt)])
        plsc.subcore_barrier()
        pltpu.sync_copy(tab_hbm.at[c], tab_v)
        coff = c * n_src
        bbase = s * nblk

        def fire_loads(b, blk):
            pltpu.async_copy(sidx_hbm.at[blk], sidx_v.at[b], lsem[b])
            pltpu.async_copy(didx_hbm.at[blk], didx_v.at[b], lsem[b])
            pltpu.async_copy(ev_hbm.at[blk], ev_v.at[b], lsem[b])

        def drain_loads(b, blk):
            pltpu.make_async_copy(sidx_hbm.at[blk], sidx_v.at[b], lsem[b]).wait()
            pltpu.make_async_copy(didx_hbm.at[blk], didx_v.at[b], lsem[b]).wait()
            pltpu.make_async_copy(ev_hbm.at[blk], ev_v.at[b], lsem[b]).wait()

        def offset_sidx(b):
            for kk in range(gp):
                for t in range(CH // 16):
                    sl = pl.ds(t * 16, 16)
                    sidx_v[b, kk, sl] = sidx_v[b, kk, sl] + coff

        def fire_gathers(b):
            for kk in range(gp):
                pltpu.async_copy(x_hbm.at[sidx_v.at[b, kk]],
                                 rows_v.at[b, kk], gsem[b])

        def drain_gathers(b):
            for kk in range(gp):
                pltpu.make_async_copy(x_hbm.at[sidx_v.at[b, kk]],
                                      rows_v.at[b, kk], gsem[b]).wait()

        def drain_scatters(b):
            for kk in range(gp):
                pltpu.make_async_copy(rows_v.at[b, kk],
                                      acc.at[didx_v.at[b, kk]], ssem[b]).wait()

        def compute_and_scatter(b):
            def cbody(kk, carry):
                for g in range(CH // 16):
                    qf16 = ev_v[b, kk, pl.ds(g * 16, 16)]
                    qi16 = qf16.astype(jnp.int32)
                    fr16 = qf16 - qi16.astype(jnp.float32)
                    for l in range(16):
                        e = g * 16 + l
                        q = qi16[l]
                        f = fr16[l]
                        g0 = (tab_v[q, pl.ds(0, 16)]
                              + f * tab_v[q, pl.ds(32, 16)])
                        g1 = (tab_v[q, pl.ds(16, 16)]
                              + f * tab_v[q, pl.ds(48, 16)])
                        rows_v[b, kk, e, pl.ds(0, 16)] = (
                            rows_v[b, kk, e, pl.ds(0, 16)] * g0)
                        rows_v[b, kk, e, pl.ds(16, 16)] = (
                            rows_v[b, kk, e, pl.ds(16, 16)] * g1)
                pltpu.async_copy(rows_v.at[b, kk], acc.at[didx_v.at[b, kk]],
                                 ssem[b], add=True)
                return carry
            lax.fori_loop(0, gp, cbody, 0)

        pltpu.sync_copy(zidx_hbm, didx_v.at[1])
        for kk in range(gp):
            pltpu.sync_copy(zer_hbm.at[pl.ds(0, CH)], rows_v.at[1, kk])
        for kk in range(gp):
            pltpu.async_copy(rows_v.at[1, kk], acc.at[didx_v.at[1, kk]],
                             ssem[1], add=True)
        fire_loads(0, bbase)
        drain_loads(0, bbase)
        offset_sidx(0)
        fire_gathers(0)

        def iteration(j, b):
            bn = 1 - b
            blkn = bbase + jnp.minimum(j + 1, nblk - 1)
            drain_scatters(bn)
            fire_loads(bn, blkn)
            drain_loads(bn, blkn)
            offset_sidx(bn)
            fire_gathers(bn)
            drain_gathers(b)
            compute_and_scatter(b)

        def obody(jj, carry):
            iteration(2 * jj, 0)
            iteration(2 * jj + 1, 1)
            return carry

        lax.fori_loop(0, nblk // 2, obody, 0)
        drain_gathers(0)
        drain_scatters(1)
        plsc.subcore_barrier()
        pltpu.sync_copy(acc.at[pl.ds(s * rpt, rpt)],
                        out_hbm.at[c, pl.ds(s * rpt, rpt)])

    return k


def _cnt_kernel_make(n_dst_pad, gp):
    rpt = n_dst_pad // N_TILES
    mesh = plsc.VectorSubcoreMesh(core_axis_name="c", subcore_axis_name="s")

    @functools.partial(
        pl.kernel, mesh=mesh,
        compiler_params=pltpu.CompilerParams(use_tc_tiling_on_sc=False),
        out_type=jax.ShapeDtypeStruct((2, n_dst_pad, 8), jnp.float32),
        scratch_types=[
            pltpu.VMEM((gp, CH), jnp.int32),
            pltpu.VMEM((CH, 8), jnp.float32),
            pltpu.VMEM_SHARED((n_dst_pad, 8), jnp.float32),
            pltpu.SemaphoreType.DMA,
        ],
    )
    def k(didx_hbm, ones_hbm, zer_hbm, out_hbm, didx_v, ones_v, acc, ssem):
        c = lax.axis_index("c")
        s = lax.axis_index("s")
        pltpu.sync_copy(zer_hbm, acc.at[pl.ds(s * rpt, rpt)])
        plsc.subcore_barrier()
        pltpu.sync_copy(ones_hbm, ones_v)
        w = c * N_TILES + s
        nblk_w = E_PAD // (CH * gp) // 32
        bbase = w * nblk_w

        def body(j, carry):
            pltpu.sync_copy(didx_hbm.at[bbase + j], didx_v)
            for kk in range(gp):
                pltpu.async_copy(ones_v, acc.at[didx_v.at[kk]], ssem,
                                 add=True)
            for kk in range(gp):
                pltpu.make_async_copy(ones_v, acc.at[didx_v.at[kk]],
                                      ssem).wait()
            return carry

        lax.fori_loop(0, nblk_w, body, 0)
        plsc.subcore_barrier()
        pltpu.sync_copy(acc.at[pl.ds(s * rpt, rpt)],
                        out_hbm.at[c, pl.ds(s * rpt, rpt)])

    return k


def _lin(p, x):
    return x @ p['W'].T + p['b']


def _ln(p, x):
    mu = x.mean(-1, keepdims=True)
    var = x.var(-1, keepdims=True)
    return (x - mu) / jnp.sqrt(var + 1e-05) * p['g'] + p['b']


def _emb(p, x):
    x = jax.nn.relu(_ln(p['n1'], _lin(p['l1'], x)))
    return jax.nn.relu(_ln(p['n2'], _lin(p['l2'], x)))


def _mha(p, x):
    m, d = x.shape
    dh = d // N_HEADS
    qkv = x @ p['in_w'].T + p['in_b']
    q, k, v = jnp.split(qkv, 3, axis=-1)
    q = q.reshape(m, N_HEADS, dh).transpose(1, 0, 2)
    k = k.reshape(m, N_HEADS, dh).transpose(1, 0, 2)
    v = v.reshape(m, N_HEADS, dh).transpose(1, 0, 2)
    a = jax.nn.softmax(q @ k.transpose(0, 2, 1) / np.sqrt(dh), axis=-1)
    o = (a @ v).transpose(1, 0, 2).reshape(m, d)
    return _lin(p['out'], o)


def _block(p, x):
    h = x + _mha(p, _ln(p['sa_norm'], x))
    f = _ln(p['ff_norm'], h)
    return h + _lin(p['ff2'], jax.nn.gelu(_lin(p['ff1'], f), approximate=False))


def _mma(p, sta, dyn):
    d = sta.shape[1]
    K = _lin(p['W_k'], sta)
    Vd = _lin(p['W_v_dyn'], dyn)
    Vs = _lin(p['W_v_sta'], sta)
    S = p['Q_macro'] @ K.T / np.sqrt(d)
    Wm = jax.nn.softmax(S, axis=0)
    Wn = Wm / jnp.clip(Wm.sum(1, keepdims=True), 1e-08, None)
    Hd = _block(p['blk_dyn'], Wn @ Vd)
    Hs = _block(p['blk_sta'], Wn @ Vs)
    fb_d = Wm.T @ _lin(p['W_out_dyn'], Hd)
    fb_s = Wm.T @ _lin(p['W_out_sta'], Hs)
    return fb_d, fb_s


def _norm_var(x):
    out = x.at[:, 19].set(jnp.log1p(jnp.abs(x[:, 19])) * jnp.sign(x[:, 19]))
    cols = jnp.array([0, 7, 8, 9, 12, 14, 19, 20])
    v = out[:, cols]
    vn = jnp.clip((v - v.mean(0)) / (jnp.std(v, axis=0, ddof=1) + 1e-06), -CLIP, CLIP)
    return out.at[:, cols].set(vn)


def _norm_con(x):
    out = x.at[:, 5].set(jnp.log1p(jnp.abs(x[:, 5])) * jnp.sign(x[:, 5]))
    cols = jnp.array([0, 1, 3, 4, 5])
    v = out[:, cols]
    vn = jnp.clip((v - v.mean(0)) / (jnp.std(v, axis=0, ddof=1) + 1e-06), -CLIP, CLIP)
    return out.at[:, cols].set(vn)


def _norm_edge(e):
    return jnp.clip((e - e.mean()) / (jnp.std(e, ddof=1) + 1e-06), -CLIP, CLIP)


def _head_body(fused_ref, w1_ref, b1_ref, w2_ref, b2_ref, out_ref):
    h1 = jax.nn.relu(jnp.dot(fused_ref[...], w1_ref[...],
                             preferred_element_type=jnp.float32) + b1_ref[...])
    out_ref[...] = jnp.dot(h1, w2_ref[...],
                           preferred_element_type=jnp.float32) + b2_ref[...]


def _head_pallas(fused, p1, p2):
    n = fused.shape[0]
    blk = 2000
    out = pl.pallas_call(
        _head_body,
        grid=(n // blk,),
        in_specs=[
            pl.BlockSpec((blk, H), lambda i: (i, 0)),
            pl.BlockSpec((H, H), lambda i: (0, 0)),
            pl.BlockSpec((H,), lambda i: (0,)),
            pl.BlockSpec((H, 1), lambda i: (0, 0)),
            pl.BlockSpec((1,), lambda i: (0,)),
        ],
        out_specs=pl.BlockSpec((blk, 1), lambda i: (i, 0)),
        out_shape=jax.ShapeDtypeStruct((n, 1), jnp.float32),
    )(fused, p1['W'].T, p1['b'], p2['W'].T, p2['b'])
    return out[:, 0]


N_BINS = 128
EV_LO = -5.0
EV_STEP = 10.0 / (N_BINS - 1)


def _gate_table(p):
    w = p['W'][:, 0]
    b = p['b']
    grid = EV_LO + EV_STEP * jnp.arange(N_BINS, dtype=jnp.float32)
    t = jax.nn.sigmoid(grid[:, None] * w[None, :] + b[None, :])
    dt = jnp.concatenate([t[1:] - t[:-1], jnp.zeros((1, H), jnp.float32)], 0)
    return jnp.stack([
        jnp.concatenate([t[:, 0:32], dt[:, 0:32]], axis=1),
        jnp.concatenate([t[:, 32:64], dt[:, 32:64]], axis=1),
    ])


class _MP:
    def __init__(self, ci, vi, ev):
        pad = E_PAD - E
        ar = jnp.arange(pad, dtype=jnp.int32)
        rs = lambda a, gp: a.reshape(E_PAD // (CH * gp), gp, CH)
        qf = jnp.clip((ev - EV_LO) / EV_STEP, 0.0, N_BINS - 1.001)
        evp = jnp.concatenate([qf, jnp.full((pad,), 63.5, jnp.float32)])
        self.vi_g = rs(jnp.concatenate([vi, ar % N_VAR]), GP_C)
        self.ci_g = rs(jnp.concatenate([ci, ar % N_CON]), GP_V)
        self.vi_s = rs(jnp.concatenate([vi, N_VAR + (ar % 16)]), GP_V)
        self.ci_s = rs(jnp.concatenate([ci, N_CON + (ar % 16)]), GP_C)
        self.ev_c = rs(evp, GP_C)
        self.ev_v = rs(evp, GP_V)
        self.zidx_c = jnp.zeros((GP_C, CH), jnp.int32)
        self.zidx_v = jnp.zeros((GP_V, CH), jnp.int32)
        self.zer_v = jnp.zeros((VAR_PAD // N_TILES, 32), jnp.float32)
        self.zer_c = jnp.zeros((CON_PAD // N_TILES, 32), jnp.float32)
        self.to_con = _mp_kernel_make(N_VAR, CON_PAD, GP_C)
        self.to_var = _mp_kernel_make(N_CON, VAR_PAD, GP_V)
        ones = jnp.ones((CH, 8), jnp.float32)
        cnt_c = _cnt_kernel_make(CON_PAD, GP_C)(
            self.ci_s, ones, jnp.zeros((CON_PAD // N_TILES, 8), jnp.float32))
        cnt_v = _cnt_kernel_make(VAR_PAD, GP_V)(
            self.vi_s, ones, jnp.zeros((VAR_PAD // N_TILES, 8), jnp.float32))
        self.inv_c = 1.0 / jnp.clip(cnt_c.sum(0)[:N_CON, 0], 1.0, None)
        self.inv_v = 1.0 / jnp.clip(cnt_v.sum(0)[:N_VAR, 0], 1.0, None)

    def v2c(self, x, gate_p):
        x2 = jnp.concatenate([x[:, :32], x[:, 32:]], axis=0)
        out = self.to_con(x2, self.vi_g, self.ci_s, self.ev_c,
                          _gate_table(gate_p), self.zer_c, self.zidx_c)
        agg = jnp.concatenate([out[0, :N_CON], out[1, :N_CON]], axis=-1)
        return agg * self.inv_c[:, None]

    def c2v(self, x, gate_p):
        x2 = jnp.concatenate([x[:, :32], x[:, 32:]], axis=0)
        out = self.to_var(x2, self.ci_g, self.vi_s, self.ev_v,
                          _gate_table(gate_p), self.zer_v, self.zidx_v)
        agg = jnp.concatenate([out[0, :N_VAR], out[1, :N_VAR]], axis=-1)
        return agg * self.inv_v[:, None]


def _gcn_layer(p, vh, ch, mp):
    agg = mp.v2c(_lin(p['v2c_lin'], vh), p['v2c_gate'])
    ch_new = jax.nn.relu(_ln(p['v2c_ln'], _lin(p['v2c_upd'], jnp.concatenate([agg, ch], -1))))
    agg = mp.c2v(_lin(p['c2v_lin'], ch_new), p['c2v_gate'])
    vh_new = jax.nn.relu(_ln(p['c2v_ln'], _lin(p['c2v_upd'], jnp.concatenate([agg, vh], -1))))
    return vh_new, ch_new


def _gcn(p, vf, cf, mp):
    vh = _emb(p['var_emb'], vf)
    ch = _emb(p['con_emb'], cf)
    for lp in p['layers']:
        dv, dc = _gcn_layer(lp, vh, ch, mp)
        vh = vh + dv
        ch = ch + dc
    return vh, ch


def kernel(var_feats, con_feats, edge_index, edge_val, params):
    vf = _norm_var(var_feats)
    cf = _norm_con(con_feats)
    ev = _norm_edge(edge_val)
    ci, vi = edge_index[0], edge_index[1]
    mp = _MP(ci, vi, ev)
    vh_s, ch_s = _gcn(params['gcn_sta'], vf[:, STATIC_VAR_IDX], cf, mp)
    vh_d, ch_d = _gcn(params['gcn_dyn'], vf[:, DYNAMIC_VAR_IDX], cf, mp)
    fb_d, fb_s = _mma(params['mma'], vh_s, vh_d)
    fused = jax.nn.relu(_ln(params['fuse_ln'], _lin(params['fuse'], jnp.concatenate([vh_s + fb_s, vh_d + fb_d], -1))))
    return _head_pallas(fused, params['head1'], params['head2'])


# speedup vs baseline: 2.0794x; 1.0041x over previous
import functools

import jax
import jax.numpy as jnp
import numpy as np
from jax import lax
from jax.experimental import pallas as pl
from jax.experimental.pallas import tpu as pltpu
from jax.experimental.pallas import tpu_sc as plsc

H = 64
N_VAR = 50000
N_CON = 25000
E = 800000
N_PROBES = 16
N_HEADS = 4
CLIP = 5.0
STATIC_VAR_IDX = np.array([0, 1, 2, 3, 4, 5, 6, 19, 20])
DYNAMIC_VAR_IDX = np.array([7, 8, 9, 10, 11, 12, 13, 14, 15, 16, 17, 18])

CH = 128
GP_C = 8
GP_V = 2
N_TILES = 16
E_PAD = 819200
EP_TILE = E_PAD // N_TILES
VAR_PAD = 50048
CON_PAD = 25088
E_CNT_W = E_PAD // 32
NCH_CNT = E_CNT_W // CH


def _mp_kernel_make(n_src, n_dst_pad, gp):
    rpt = n_dst_pad // N_TILES
    nblk = EP_TILE // (CH * gp)
    mesh = plsc.VectorSubcoreMesh(core_axis_name="c", subcore_axis_name="s")

    @functools.partial(
        pl.kernel, mesh=mesh,
        compiler_params=pltpu.CompilerParams(use_tc_tiling_on_sc=False),
        out_type=jax.ShapeDtypeStruct((2, n_dst_pad, 32), jnp.float32),
        scratch_types=[
            pltpu.VMEM((2, gp, CH), jnp.int32),
            pltpu.VMEM((2, gp, CH), jnp.int32),
            pltpu.VMEM((2, gp, CH), jnp.float32),
            pltpu.VMEM((2, gp, CH, 32), jnp.float32),
            pltpu.VMEM((128, 64), jnp.float32),
            pltpu.VMEM_SHARED((n_dst_pad, 32), jnp.float32),
            pltpu.SemaphoreType.DMA,
            pltpu.SemaphoreType.DMA,
            pltpu.SemaphoreType.DMA,
            pltpu.SemaphoreType.DMA,
            pltpu.SemaphoreType.DMA,
            pltpu.SemaphoreType.DMA,
        ],
    )
    def k(x_hbm, sidx_hbm, didx_hbm, ev_hbm, tab_hbm, zer_hbm, zidx_hbm,
          out_hbm, sidx_v, didx_v, ev_v, rows_v, tab_v, acc,
          lsem0, lsem1, gsem0, gsem1, ssem0, ssem1):
        lsem = (lsem0, lsem1)
        gsem = (gsem0, gsem1)
        ssem = (ssem0, ssem1)
        c = lax.axis_index("c")
        s = lax.axis_index("s")
        pltpu.sync_copy(zer_hbm, acc.at[pl.ds(s * rpt, rpt)])
        plsc.subcore_barrier()
        pltpu.sync_copy(tab_hbm.at[c], tab_v)
        bbase = s * nblk

        def fire_loads(b, blk):
            pltpu.async_copy(sidx_hbm.at[blk], sidx_v.at[b], lsem[b])
            pltpu.async_copy(didx_hbm.at[blk], didx_v.at[b], lsem[b])
            pltpu.async_copy(ev_hbm.at[blk], ev_v.at[b], lsem[b])

        def drain_loads(b, blk):
            pltpu.make_async_copy(sidx_hbm.at[blk], sidx_v.at[b], lsem[b]).wait()
            pltpu.make_async_copy(didx_hbm.at[blk], didx_v.at[b], lsem[b]).wait()
            pltpu.make_async_copy(ev_hbm.at[blk], ev_v.at[b], lsem[b]).wait()

        def offset_sidx(b):
            for kk in range(gp):
                for t in range(CH // 16):
                    sl = pl.ds(t * 16, 16)
                    sidx_v[b, kk, sl] = sidx_v[b, kk, sl] * 2 + c

        def fire_gathers(b):
            for kk in range(gp):
                pltpu.async_copy(x_hbm.at[sidx_v.at[b, kk]],
                                 rows_v.at[b, kk], gsem[b])

        def drain_gathers(b):
            for kk in range(gp):
                pltpu.make_async_copy(x_hbm.at[sidx_v.at[b, kk]],
                                      rows_v.at[b, kk], gsem[b]).wait()

        def drain_scatters(b):
            for kk in range(gp):
                pltpu.make_async_copy(rows_v.at[b, kk],
                                      acc.at[didx_v.at[b, kk]], ssem[b]).wait()

        def compute_and_scatter(b):
            def cbody(kk, carry):
                for g in range(CH // 16):
                    qf16 = ev_v[b, kk, pl.ds(g * 16, 16)]
                    qi16 = qf16.astype(jnp.int32)
                    fr16 = qf16 - qi16.astype(jnp.float32)
                    for l in range(16):
                        e = g * 16 + l
                        q = qi16[l]
                        f = fr16[l]
                        g0 = (tab_v[q, pl.ds(0, 16)]
                              + f * tab_v[q, pl.ds(32, 16)])
                        g1 = (tab_v[q, pl.ds(16, 16)]
                              + f * tab_v[q, pl.ds(48, 16)])
                        rows_v[b, kk, e, pl.ds(0, 16)] = (
                            rows_v[b, kk, e, pl.ds(0, 16)] * g0)
                        rows_v[b, kk, e, pl.ds(16, 16)] = (
                            rows_v[b, kk, e, pl.ds(16, 16)] * g1)
                pltpu.async_copy(rows_v.at[b, kk], acc.at[didx_v.at[b, kk]],
                                 ssem[b], add=True)
                return carry
            lax.fori_loop(0, gp, cbody, 0)

        pltpu.sync_copy(zidx_hbm, didx_v.at[1])
        for kk in range(gp):
            pltpu.sync_copy(zer_hbm.at[pl.ds(0, CH)], rows_v.at[1, kk])
        for kk in range(gp):
            pltpu.async_copy(rows_v.at[1, kk], acc.at[didx_v.at[1, kk]],
                             ssem[1], add=True)
        fire_loads(0, bbase)
        drain_loads(0, bbase)
        offset_sidx(0)
        fire_gathers(0)

        def iteration(j, b):
            bn = 1 - b
            blkn = bbase + jnp.minimum(j + 1, nblk - 1)
            drain_scatters(bn)
            fire_loads(bn, blkn)
            drain_loads(bn, blkn)
            offset_sidx(bn)
            fire_gathers(bn)
            drain_gathers(b)
            compute_and_scatter(b)

        def obody(jj, carry):
            iteration(2 * jj, 0)
            iteration(2 * jj + 1, 1)
            return carry

        lax.fori_loop(0, nblk // 2, obody, 0)
        drain_gathers(0)
        drain_scatters(1)
        plsc.subcore_barrier()
        pltpu.sync_copy(acc.at[pl.ds(s * rpt, rpt)],
                        out_hbm.at[c, pl.ds(s * rpt, rpt)])

    return k


def _cnt_kernel_make(n_dst_pad, gp):
    rpt = n_dst_pad // N_TILES
    mesh = plsc.VectorSubcoreMesh(core_axis_name="c", subcore_axis_name="s")

    @functools.partial(
        pl.kernel, mesh=mesh,
        compiler_params=pltpu.CompilerParams(use_tc_tiling_on_sc=False),
        out_type=jax.ShapeDtypeStruct((2, n_dst_pad, 8), jnp.float32),
        scratch_types=[
            pltpu.VMEM((gp, CH), jnp.int32),
            pltpu.VMEM((CH, 8), jnp.float32),
            pltpu.VMEM_SHARED((n_dst_pad, 8), jnp.float32),
            pltpu.SemaphoreType.DMA,
        ],
    )
    def k(didx_hbm, ones_hbm, zer_hbm, out_hbm, didx_v, ones_v, acc, ssem):
        c = lax.axis_index("c")
        s = lax.axis_index("s")
        pltpu.sync_copy(zer_hbm, acc.at[pl.ds(s * rpt, rpt)])
        plsc.subcore_barrier()
        pltpu.sync_copy(ones_hbm, ones_v)
        w = c * N_TILES + s
        nblk_w = E_PAD // (CH * gp) // 32
        bbase = w * nblk_w

        def body(j, carry):
            pltpu.sync_copy(didx_hbm.at[bbase + j], didx_v)
            for kk in range(gp):
                pltpu.async_copy(ones_v, acc.at[didx_v.at[kk]], ssem,
                                 add=True)
            for kk in range(gp):
                pltpu.make_async_copy(ones_v, acc.at[didx_v.at[kk]],
                                      ssem).wait()
            return carry

        lax.fori_loop(0, nblk_w, body, 0)
        plsc.subcore_barrier()
        pltpu.sync_copy(acc.at[pl.ds(s * rpt, rpt)],
                        out_hbm.at[c, pl.ds(s * rpt, rpt)])

    return k


def _lin(p, x):
    return x @ p['W'].T + p['b']


def _ln(p, x):
    mu = x.mean(-1, keepdims=True)
    var = x.var(-1, keepdims=True)
    return (x - mu) / jnp.sqrt(var + 1e-05) * p['g'] + p['b']


def _emb(p, x):
    x = jax.nn.relu(_ln(p['n1'], _lin(p['l1'], x)))
    return jax.nn.relu(_ln(p['n2'], _lin(p['l2'], x)))


def _mha(p, x):
    m, d = x.shape
    dh = d // N_HEADS
    qkv = x @ p['in_w'].T + p['in_b']
    q, k, v = jnp.split(qkv, 3, axis=-1)
    q = q.reshape(m, N_HEADS, dh).transpose(1, 0, 2)
    k = k.reshape(m, N_HEADS, dh).transpose(1, 0, 2)
    v = v.reshape(m, N_HEADS, dh).transpose(1, 0, 2)
    a = jax.nn.softmax(q @ k.transpose(0, 2, 1) / np.sqrt(dh), axis=-1)
    o = (a @ v).transpose(1, 0, 2).reshape(m, d)
    return _lin(p['out'], o)


def _block(p, x):
    h = x + _mha(p, _ln(p['sa_norm'], x))
    f = _ln(p['ff_norm'], h)
    return h + _lin(p['ff2'], jax.nn.gelu(_lin(p['ff1'], f), approximate=False))


def _mma(p, sta, dyn):
    d = sta.shape[1]
    K = _lin(p['W_k'], sta)
    Vd = _lin(p['W_v_dyn'], dyn)
    Vs = _lin(p['W_v_sta'], sta)
    S = p['Q_macro'] @ K.T / np.sqrt(d)
    Wm = jax.nn.softmax(S, axis=0)
    Wn = Wm / jnp.clip(Wm.sum(1, keepdims=True), 1e-08, None)
    Hd = _block(p['blk_dyn'], Wn @ Vd)
    Hs = _block(p['blk_sta'], Wn @ Vs)
    fb_d = Wm.T @ _lin(p['W_out_dyn'], Hd)
    fb_s = Wm.T @ _lin(p['W_out_sta'], Hs)
    return fb_d, fb_s


def _norm_var(x):
    out = x.at[:, 19].set(jnp.log1p(jnp.abs(x[:, 19])) * jnp.sign(x[:, 19]))
    cols = jnp.array([0, 7, 8, 9, 12, 14, 19, 20])
    v = out[:, cols]
    vn = jnp.clip((v - v.mean(0)) / (jnp.std(v, axis=0, ddof=1) + 1e-06), -CLIP, CLIP)
    return out.at[:, cols].set(vn)


def _norm_con(x):
    out = x.at[:, 5].set(jnp.log1p(jnp.abs(x[:, 5])) * jnp.sign(x[:, 5]))
    cols = jnp.array([0, 1, 3, 4, 5])
    v = out[:, cols]
    vn = jnp.clip((v - v.mean(0)) / (jnp.std(v, axis=0, ddof=1) + 1e-06), -CLIP, CLIP)
    return out.at[:, cols].set(vn)


def _norm_edge(e):
    return jnp.clip((e - e.mean()) / (jnp.std(e, ddof=1) + 1e-06), -CLIP, CLIP)


def _head_body(fused_ref, w1_ref, b1_ref, w2_ref, b2_ref, out_ref):
    h1 = jax.nn.relu(jnp.dot(fused_ref[...], w1_ref[...],
                             preferred_element_type=jnp.float32) + b1_ref[...])
    out_ref[...] = jnp.dot(h1, w2_ref[...],
                           preferred_element_type=jnp.float32) + b2_ref[...]


def _head_pallas(fused, p1, p2):
    n = fused.shape[0]
    blk = 2000
    out = pl.pallas_call(
        _head_body,
        grid=(n // blk,),
        in_specs=[
            pl.BlockSpec((blk, H), lambda i: (i, 0)),
            pl.BlockSpec((H, H), lambda i: (0, 0)),
            pl.BlockSpec((H,), lambda i: (0,)),
            pl.BlockSpec((H, 1), lambda i: (0, 0)),
            pl.BlockSpec((1,), lambda i: (0,)),
        ],
        out_specs=pl.BlockSpec((blk, 1), lambda i: (i, 0)),
        out_shape=jax.ShapeDtypeStruct((n, 1), jnp.float32),
    )(fused, p1['W'].T, p1['b'], p2['W'].T, p2['b'])
    return out[:, 0]


N_BINS = 128
EV_LO = -5.0
EV_STEP = 10.0 / (N_BINS - 1)


def _gate_table(p):
    w = p['W'][:, 0]
    b = p['b']
    grid = EV_LO + EV_STEP * jnp.arange(N_BINS, dtype=jnp.float32)
    t = jax.nn.sigmoid(grid[:, None] * w[None, :] + b[None, :])
    dt = jnp.concatenate([t[1:] - t[:-1], jnp.zeros((1, H), jnp.float32)], 0)
    return jnp.stack([
        jnp.concatenate([t[:, 0:32], dt[:, 0:32]], axis=1),
        jnp.concatenate([t[:, 32:64], dt[:, 32:64]], axis=1),
    ])


class _MP:
    def __init__(self, ci, vi, ev):
        pad = E_PAD - E
        ar = jnp.arange(pad, dtype=jnp.int32)
        rs = lambda a, gp: a.reshape(E_PAD // (CH * gp), gp, CH)
        qf = jnp.clip((ev - EV_LO) / EV_STEP, 0.0, N_BINS - 1.001)
        evp = jnp.concatenate([qf, jnp.full((pad,), 63.5, jnp.float32)])
        self.vi_g = rs(jnp.concatenate([vi, ar % N_VAR]), GP_C)
        self.ci_g = rs(jnp.concatenate([ci, ar % N_CON]), GP_V)
        self.vi_s = rs(jnp.concatenate([vi, N_VAR + (ar % 16)]), GP_V)
        self.ci_s = rs(jnp.concatenate([ci, N_CON + (ar % 16)]), GP_C)
        self.ev_c = rs(evp, GP_C)
        self.ev_v = rs(evp, GP_V)
        self.zidx_c = jnp.zeros((GP_C, CH), jnp.int32)
        self.zidx_v = jnp.zeros((GP_V, CH), jnp.int32)
        self.zer_v = jnp.zeros((VAR_PAD // N_TILES, 32), jnp.float32)
        self.zer_c = jnp.zeros((CON_PAD // N_TILES, 32), jnp.float32)
        self.to_con = _mp_kernel_make(N_VAR, CON_PAD, GP_C)
        self.to_var = _mp_kernel_make(N_CON, VAR_PAD, GP_V)
        ones = jnp.ones((CH, 8), jnp.float32)
        cnt_c = _cnt_kernel_make(CON_PAD, GP_C)(
            self.ci_s, ones, jnp.zeros((CON_PAD // N_TILES, 8), jnp.float32))
        cnt_v = _cnt_kernel_make(VAR_PAD, GP_V)(
            self.vi_s, ones, jnp.zeros((VAR_PAD // N_TILES, 8), jnp.float32))
        self.inv_c = 1.0 / jnp.clip(cnt_c.sum(0)[:N_CON, 0], 1.0, None)
        self.inv_v = 1.0 / jnp.clip(cnt_v.sum(0)[:N_VAR, 0], 1.0, None)

    def v2c(self, x, gate_p):
        x2 = x.reshape(2 * N_VAR, 32)
        out = self.to_con(x2, self.vi_g, self.ci_s, self.ev_c,
                          _gate_table(gate_p), self.zer_c, self.zidx_c)
        agg = jnp.concatenate([out[0, :N_CON], out[1, :N_CON]], axis=-1)
        return agg * self.inv_c[:, None]

    def c2v(self, x, gate_p):
        x2 = x.reshape(2 * N_CON, 32)
        out = self.to_var(x2, self.ci_g, self.vi_s, self.ev_v,
                          _gate_table(gate_p), self.zer_v, self.zidx_v)
        agg = jnp.concatenate([out[0, :N_VAR], out[1, :N_VAR]], axis=-1)
        return agg * self.inv_v[:, None]


def _gcn_layer(p, vh, ch, mp):
    agg = mp.v2c(_lin(p['v2c_lin'], vh), p['v2c_gate'])
    ch_new = jax.nn.relu(_ln(p['v2c_ln'], _lin(p['v2c_upd'], jnp.concatenate([agg, ch], -1))))
    agg = mp.c2v(_lin(p['c2v_lin'], ch_new), p['c2v_gate'])
    vh_new = jax.nn.relu(_ln(p['c2v_ln'], _lin(p['c2v_upd'], jnp.concatenate([agg, vh], -1))))
    return vh_new, ch_new


def _gcn(p, vf, cf, mp):
    vh = _emb(p['var_emb'], vf)
    ch = _emb(p['con_emb'], cf)
    for lp in p['layers']:
        dv, dc = _gcn_layer(lp, vh, ch, mp)
        vh = vh + dv
        ch = ch + dc
    return vh, ch


def kernel(var_feats, con_feats, edge_index, edge_val, params):
    vf = _norm_var(var_feats)
    cf = _norm_con(con_feats)
    ev = _norm_edge(edge_val)
    ci, vi = edge_index[0], edge_index[1]
    mp = _MP(ci, vi, ev)
    vh_s, ch_s = _gcn(params['gcn_sta'], vf[:, STATIC_VAR_IDX], cf, mp)
    vh_d, ch_d = _gcn(params['gcn_dyn'], vf[:, DYNAMIC_VAR_IDX], cf, mp)
    fb_d, fb_s = _mma(params['mma'], vh_s, vh_d)
    fused = jax.nn.relu(_ln(params['fuse_ln'], _lin(params['fuse'], jnp.concatenate([vh_s + fb_s, vh_d + fb_d], -1))))
    return _head_pallas(fused, params['head1'], params['head2'])


# speedup vs baseline: 2.0814x; 1.0010x over previous
import functools

import jax
import jax.numpy as jnp
import numpy as np
from jax import lax
from jax.experimental import pallas as pl
from jax.experimental.pallas import tpu as pltpu
from jax.experimental.pallas import tpu_sc as plsc

H = 64
N_VAR = 50000
N_CON = 25000
E = 800000
N_PROBES = 16
N_HEADS = 4
CLIP = 5.0
STATIC_VAR_IDX = np.array([0, 1, 2, 3, 4, 5, 6, 19, 20])
DYNAMIC_VAR_IDX = np.array([7, 8, 9, 10, 11, 12, 13, 14, 15, 16, 17, 18])

CH = 128
GP_C = 8
GP_V = 2
N_TILES = 16
E_PAD = 819200
EP_TILE = E_PAD // N_TILES
VAR_PAD = 50048
CON_PAD = 25088


def _mp_kernel_make(n_src, n_dst_pad, gp):
    rpt = n_dst_pad // N_TILES
    nblk = EP_TILE // (CH * gp)
    mesh = plsc.VectorSubcoreMesh(core_axis_name="c", subcore_axis_name="s")

    @functools.partial(
        pl.kernel, mesh=mesh,
        compiler_params=pltpu.CompilerParams(use_tc_tiling_on_sc=False),
        out_type=jax.ShapeDtypeStruct((2, n_dst_pad, 32), jnp.float32),
        scratch_types=[
            pltpu.VMEM((2, gp, CH), jnp.int32),
            pltpu.VMEM((2, gp, CH), jnp.int32),
            pltpu.VMEM((2, gp, CH), jnp.float32),
            pltpu.VMEM((2, gp, CH, 32), jnp.float32),
            pltpu.VMEM((128, 64), jnp.float32),
            pltpu.VMEM_SHARED((n_dst_pad, 32), jnp.float32),
            pltpu.SemaphoreType.DMA,
            pltpu.SemaphoreType.DMA,
            pltpu.SemaphoreType.DMA,
            pltpu.SemaphoreType.DMA,
            pltpu.SemaphoreType.DMA,
            pltpu.SemaphoreType.DMA,
        ],
    )
    def k(x_hbm, sidx_hbm, didx_hbm, ev_hbm, tab_hbm, zer_hbm, zidx_hbm,
          out_hbm, sidx_v, didx_v, ev_v, rows_v, tab_v, acc,
          lsem0, lsem1, gsem0, gsem1, ssem0, ssem1):
        lsem = (lsem0, lsem1)
        gsem = (gsem0, gsem1)
        ssem = (ssem0, ssem1)
        c = lax.axis_index("c")
        s = lax.axis_index("s")
        pltpu.sync_copy(zer_hbm, acc.at[pl.ds(s * rpt, rpt)])
        plsc.subcore_barrier()
        pltpu.sync_copy(tab_hbm.at[c], tab_v)
        bbase = s * nblk

        def fire_loads(b, blk):
            pltpu.async_copy(sidx_hbm.at[blk], sidx_v.at[b], lsem[b])
            pltpu.async_copy(didx_hbm.at[blk], didx_v.at[b], lsem[b])
            pltpu.async_copy(ev_hbm.at[blk], ev_v.at[b], lsem[b])

        def drain_loads(b, blk):
            pltpu.make_async_copy(sidx_hbm.at[blk], sidx_v.at[b], lsem[b]).wait()
            pltpu.make_async_copy(didx_hbm.at[blk], didx_v.at[b], lsem[b]).wait()
            pltpu.make_async_copy(ev_hbm.at[blk], ev_v.at[b], lsem[b]).wait()

        def offset_sidx(b):
            for kk in range(gp):
                for t in range(CH // 16):
                    sl = pl.ds(t * 16, 16)
                    sidx_v[b, kk, sl] = sidx_v[b, kk, sl] * 2 + c

        def fire_gathers(b):
            for kk in range(gp):
                pltpu.async_copy(x_hbm.at[sidx_v.at[b, kk]],
                                 rows_v.at[b, kk], gsem[b])

        def drain_gathers(b):
            for kk in range(gp):
                pltpu.make_async_copy(x_hbm.at[sidx_v.at[b, kk]],
                                      rows_v.at[b, kk], gsem[b]).wait()

        def drain_scatters(b):
            for kk in range(gp):
                pltpu.make_async_copy(rows_v.at[b, kk],
                                      acc.at[didx_v.at[b, kk]], ssem[b]).wait()

        def compute_and_scatter(b):
            def cbody(kk, carry):
                for g in range(CH // 16):
                    qf16 = ev_v[b, kk, pl.ds(g * 16, 16)]
                    qi16 = qf16.astype(jnp.int32)
                    fr16 = qf16 - qi16.astype(jnp.float32)
                    for l in range(16):
                        e = g * 16 + l
                        q = qi16[l]
                        f = fr16[l]
                        g0 = (tab_v[q, pl.ds(0, 16)]
                              + f * tab_v[q, pl.ds(32, 16)])
                        g1 = (tab_v[q, pl.ds(16, 16)]
                              + f * tab_v[q, pl.ds(48, 16)])
                        rows_v[b, kk, e, pl.ds(0, 16)] = (
                            rows_v[b, kk, e, pl.ds(0, 16)] * g0)
                        rows_v[b, kk, e, pl.ds(16, 16)] = (
                            rows_v[b, kk, e, pl.ds(16, 16)] * g1)
                pltpu.async_copy(rows_v.at[b, kk], acc.at[didx_v.at[b, kk]],
                                 ssem[b], add=True)
                return carry
            lax.fori_loop(0, gp, cbody, 0)

        pltpu.sync_copy(zidx_hbm, didx_v.at[1])
        for kk in range(gp):
            pltpu.sync_copy(zer_hbm.at[pl.ds(0, CH)], rows_v.at[1, kk])
        for kk in range(gp):
            pltpu.async_copy(rows_v.at[1, kk], acc.at[didx_v.at[1, kk]],
                             ssem[1], add=True)
        fire_loads(0, bbase)
        drain_loads(0, bbase)
        offset_sidx(0)
        fire_gathers(0)

        def iteration(j, b):
            bn = 1 - b
            blkn = bbase + jnp.minimum(j + 1, nblk - 1)
            drain_scatters(bn)
            fire_loads(bn, blkn)
            drain_loads(bn, blkn)
            offset_sidx(bn)
            fire_gathers(bn)
            drain_gathers(b)
            compute_and_scatter(b)

        def obody(jj, carry):
            iteration(2 * jj, 0)
            iteration(2 * jj + 1, 1)
            return carry

        lax.fori_loop(0, nblk // 2, obody, 0)
        drain_gathers(0)
        drain_scatters(1)
        plsc.subcore_barrier()
        pltpu.sync_copy(acc.at[pl.ds(s * rpt, rpt)],
                        out_hbm.at[c, pl.ds(s * rpt, rpt)])

    return k


def _cnt_kernel_make(n_dst_pad, gp):
    rpt = n_dst_pad // N_TILES
    mesh = plsc.VectorSubcoreMesh(core_axis_name="c", subcore_axis_name="s")

    @functools.partial(
        pl.kernel, mesh=mesh,
        compiler_params=pltpu.CompilerParams(use_tc_tiling_on_sc=False),
        out_type=jax.ShapeDtypeStruct((2, n_dst_pad, 8), jnp.float32),
        scratch_types=[
            pltpu.VMEM((gp, CH), jnp.int32),
            pltpu.VMEM((CH, 8), jnp.float32),
            pltpu.VMEM_SHARED((n_dst_pad, 8), jnp.float32),
            pltpu.SemaphoreType.DMA,
        ],
    )
    def k(didx_hbm, ones_hbm, zer_hbm, out_hbm, didx_v, ones_v, acc, ssem):
        c = lax.axis_index("c")
        s = lax.axis_index("s")
        pltpu.sync_copy(zer_hbm, acc.at[pl.ds(s * rpt, rpt)])
        plsc.subcore_barrier()
        pltpu.sync_copy(ones_hbm, ones_v)
        w = c * N_TILES + s
        nblk_w = E_PAD // (CH * gp) // 32
        bbase = w * nblk_w

        def body(j, carry):
            pltpu.sync_copy(didx_hbm.at[bbase + j], didx_v)
            for kk in range(gp):
                pltpu.async_copy(ones_v, acc.at[didx_v.at[kk]], ssem,
                                 add=True)
            for kk in range(gp):
                pltpu.make_async_copy(ones_v, acc.at[didx_v.at[kk]],
                                      ssem).wait()
            return carry

        lax.fori_loop(0, nblk_w, body, 0)
        plsc.subcore_barrier()
        pltpu.sync_copy(acc.at[pl.ds(s * rpt, rpt)],
                        out_hbm.at[c, pl.ds(s * rpt, rpt)])

    return k


def _lin(p, x):
    return x @ p['W'].T + p['b']


def _ln(p, x):
    mu = x.mean(-1, keepdims=True)
    var = x.var(-1, keepdims=True)
    return (x - mu) / jnp.sqrt(var + 1e-05) * p['g'] + p['b']


def _emb(p, x):
    x = jax.nn.relu(_ln(p['n1'], _lin(p['l1'], x)))
    return jax.nn.relu(_ln(p['n2'], _lin(p['l2'], x)))


def _mha(p, x):
    m, d = x.shape
    dh = d // N_HEADS
    qkv = x @ p['in_w'].T + p['in_b']
    q, k, v = jnp.split(qkv, 3, axis=-1)
    q = q.reshape(m, N_HEADS, dh).transpose(1, 0, 2)
    k = k.reshape(m, N_HEADS, dh).transpose(1, 0, 2)
    v = v.reshape(m, N_HEADS, dh).transpose(1, 0, 2)
    a = jax.nn.softmax(q @ k.transpose(0, 2, 1) / np.sqrt(dh), axis=-1)
    o = (a @ v).transpose(1, 0, 2).reshape(m, d)
    return _lin(p['out'], o)


def _block(p, x):
    h = x + _mha(p, _ln(p['sa_norm'], x))
    f = _ln(p['ff_norm'], h)
    return h + _lin(p['ff2'], jax.nn.gelu(_lin(p['ff1'], f), approximate=False))


def _mma(p, sta, dyn):
    d = sta.shape[1]
    K = _lin(p['W_k'], sta)
    Vd = _lin(p['W_v_dyn'], dyn)
    Vs = _lin(p['W_v_sta'], sta)
    S = p['Q_macro'] @ K.T / np.sqrt(d)
    Wm = jax.nn.softmax(S, axis=0)
    Wn = Wm / jnp.clip(Wm.sum(1, keepdims=True), 1e-08, None)
    Hd = _block(p['blk_dyn'], Wn @ Vd)
    Hs = _block(p['blk_sta'], Wn @ Vs)
    fb_d = Wm.T @ _lin(p['W_out_dyn'], Hd)
    fb_s = Wm.T @ _lin(p['W_out_sta'], Hs)
    return fb_d, fb_s


def _norm_var(x):
    out = x.at[:, 19].set(jnp.log1p(jnp.abs(x[:, 19])) * jnp.sign(x[:, 19]))
    cols = jnp.array([0, 7, 8, 9, 12, 14, 19, 20])
    v = out[:, cols]
    vn = jnp.clip((v - v.mean(0)) / (jnp.std(v, axis=0, ddof=1) + 1e-06), -CLIP, CLIP)
    return out.at[:, cols].set(vn)


def _norm_con(x):
    out = x.at[:, 5].set(jnp.log1p(jnp.abs(x[:, 5])) * jnp.sign(x[:, 5]))
    cols = jnp.array([0, 1, 3, 4, 5])
    v = out[:, cols]
    vn = jnp.clip((v - v.mean(0)) / (jnp.std(v, axis=0, ddof=1) + 1e-06), -CLIP, CLIP)
    return out.at[:, cols].set(vn)


def _norm_edge(e):
    return jnp.clip((e - e.mean()) / (jnp.std(e, ddof=1) + 1e-06), -CLIP, CLIP)


def _head_body(fused_ref, w1_ref, b1_ref, w2_ref, b2_ref, out_ref):
    h1 = jax.nn.relu(jnp.dot(fused_ref[...], w1_ref[...],
                             preferred_element_type=jnp.float32) + b1_ref[...])
    out_ref[...] = jnp.dot(h1, w2_ref[...],
                           preferred_element_type=jnp.float32) + b2_ref[...]


def _head_pallas(fused, p1, p2):
    n = fused.shape[0]
    blk = 2000
    out = pl.pallas_call(
        _head_body,
        grid=(n // blk,),
        in_specs=[
            pl.BlockSpec((blk, H), lambda i: (i, 0)),
            pl.BlockSpec((H, H), lambda i: (0, 0)),
            pl.BlockSpec((H,), lambda i: (0,)),
            pl.BlockSpec((H, 1), lambda i: (0, 0)),
            pl.BlockSpec((1,), lambda i: (0,)),
        ],
        out_specs=pl.BlockSpec((blk, 1), lambda i: (i, 0)),
        out_shape=jax.ShapeDtypeStruct((n, 1), jnp.float32),
    )(fused, p1['W'].T, p1['b'], p2['W'].T, p2['b'])
    return out[:, 0]


N_BINS = 128
EV_LO = -5.0
EV_STEP = 10.0 / (N_BINS - 1)


def _gate_table(p):
    w = p['W'][:, 0]
    b = p['b']
    grid = EV_LO + EV_STEP * jnp.arange(N_BINS, dtype=jnp.float32)
    t = jax.nn.sigmoid(grid[:, None] * w[None, :] + b[None, :])
    dt = jnp.concatenate([t[1:] - t[:-1], jnp.zeros((1, H), jnp.float32)], 0)
    return jnp.stack([
        jnp.concatenate([t[:, 0:32], dt[:, 0:32]], axis=1),
        jnp.concatenate([t[:, 32:64], dt[:, 32:64]], axis=1),
    ])


class _MP:
    def __init__(self, ci, vi, ev):
        pad = E_PAD - E
        ar = jnp.arange(pad, dtype=jnp.int32)
        rs = lambda a, gp: a.reshape(E_PAD // (CH * gp), gp, CH)
        qf = jnp.clip((ev - EV_LO) / EV_STEP, 0.0, N_BINS - 1.001)
        evp = jnp.concatenate([qf, jnp.full((pad,), 63.5, jnp.float32)])
        self.vi_g = rs(jnp.concatenate([vi, ar % N_VAR]), GP_C)
        self.ci_g = rs(jnp.concatenate([ci, ar % N_CON]), GP_V)
        self.vi_s = rs(jnp.concatenate([vi, N_VAR + (ar % 16)]), GP_V)
        self.ci_s = rs(jnp.concatenate([ci, N_CON + (ar % 16)]), GP_C)
        self.ev_c = rs(evp, GP_C)
        self.ev_v = rs(evp, GP_V)
        self.zidx_c = jnp.zeros((GP_C, CH), jnp.int32)
        self.zidx_v = jnp.zeros((GP_V, CH), jnp.int32)
        self.zer_v = jnp.zeros((VAR_PAD // N_TILES, 32), jnp.float32)
        self.zer_c = jnp.zeros((CON_PAD // N_TILES, 32), jnp.float32)
        self.to_con = _mp_kernel_make(N_VAR, CON_PAD, GP_C)
        self.to_var = _mp_kernel_make(N_CON, VAR_PAD, GP_V)
        ones = jnp.ones((CH, 8), jnp.float32)
        cnt_c = _cnt_kernel_make(CON_PAD, GP_C)(
            self.ci_s, ones, jnp.zeros((CON_PAD // N_TILES, 8), jnp.float32))
        cnt_v = _cnt_kernel_make(VAR_PAD, GP_V)(
            self.vi_s, ones, jnp.zeros((VAR_PAD // N_TILES, 8), jnp.float32))
        self.inv_c = 1.0 / jnp.clip(cnt_c.sum(0)[:N_CON, 0], 1.0, None)
        self.inv_v = 1.0 / jnp.clip(cnt_v.sum(0)[:N_VAR, 0], 1.0, None)

    def v2c(self, x, gate_p):
        x2 = x.reshape(2 * N_VAR, 32)
        out = self.to_con(x2, self.vi_g, self.ci_s, self.ev_c,
                          _gate_table(gate_p), self.zer_c, self.zidx_c)
        agg = jnp.concatenate([out[0, :N_CON], out[1, :N_CON]], axis=-1)
        return agg * self.inv_c[:, None]

    def c2v(self, x, gate_p):
        x2 = x.reshape(2 * N_CON, 32)
        out = self.to_var(x2, self.ci_g, self.vi_s, self.ev_v,
                          _gate_table(gate_p), self.zer_v, self.zidx_v)
        agg = jnp.concatenate([out[0, :N_VAR], out[1, :N_VAR]], axis=-1)
        return agg * self.inv_v[:, None]


def _gcn_layer(p, vh, ch, mp):
    agg = mp.v2c(_lin(p['v2c_lin'], vh), p['v2c_gate'])
    ch_new = jax.nn.relu(_ln(p['v2c_ln'], _lin(p['v2c_upd'], jnp.concatenate([agg, ch], -1))))
    agg = mp.c2v(_lin(p['c2v_lin'], ch_new), p['c2v_gate'])
    vh_new = jax.nn.relu(_ln(p['c2v_ln'], _lin(p['c2v_upd'], jnp.concatenate([agg, vh], -1))))
    return vh_new, ch_new


def _gcn(p, vf, cf, mp):
    vh = _emb(p['var_emb'], vf)
    ch = _emb(p['con_emb'], cf)
    for lp in p['layers']:
        dv, dc = _gcn_layer(lp, vh, ch, mp)
        vh = vh + dv
        ch = ch + dc
    return vh, ch


def kernel(var_feats, con_feats, edge_index, edge_val, params):
    vf = _norm_var(var_feats)
    cf = _norm_con(con_feats)
    ev = _norm_edge(edge_val)
    ci, vi = edge_index[0], edge_index[1]
    mp = _MP(ci, vi, ev)
    vh_s, ch_s = _gcn(params['gcn_sta'], vf[:, STATIC_VAR_IDX], cf, mp)
    vh_d, ch_d = _gcn(params['gcn_dyn'], vf[:, DYNAMIC_VAR_IDX], cf, mp)
    fb_d, fb_s = _mma(params['mma'], vh_s, vh_d)
    fused = jax.nn.relu(_ln(params['fuse_ln'], _lin(params['fuse'], jnp.concatenate([vh_s + fb_s, vh_d + fb_d], -1))))
    return _head_pallas(fused, params['head1'], params['head2'])
